# baseline (speedup 1.0000x reference)
import jax
import jax.numpy as jnp
import numpy as np
from jax import lax
from jax.experimental import pallas as pl
from jax.experimental.pallas import tpu as pltpu
from jax.experimental.pallas import tpu_sc as plsc

N_NODES = 10000
E_EDGES = 320000
D = 128
HD = D // 2
E_BLK = 640

CH = 4
E_C = E_EDGES // CH
NEB_C = E_C // E_BLK
K = 128
NBLK = E_EDGES // K
NBLK_C = E_C // K
NW = 32
NSUB = 16
A_STEPS = (NBLK_C + NW - 1) // NW
C_STEPS = (NBLK_C + NSUB - 1) // NSUB
N_PAD = 10240
STRIPE = N_PAD // NSUB

_MESH = plsc.VectorSubcoreMesh(core_axis_name="c", subcore_axis_name="s")

_HI = np.uint32(0xFFFF0000)


def _dotT(a, w):
    return lax.dot_general(a, w, (((1,), (1,)), ((), ())),
                           preferred_element_type=jnp.float32)


def _pack2(lo, hi):
    lo_r = lo.astype(jnp.bfloat16).astype(jnp.float32)
    hi_r = hi.astype(jnp.bfloat16).astype(jnp.float32)
    lo_u = lax.shift_right_logical(
        lax.bitcast_convert_type(lo_r, jnp.uint32), np.uint32(16))
    hi_u = lax.bitcast_convert_type(hi_r, jnp.uint32) & _HI
    return lo_u | hi_u


def _unpack_lo(u):
    return lax.bitcast_convert_type(
        lax.shift_left(u, np.uint32(16)), jnp.float32)


def _unpack_hi(u):
    return lax.bitcast_convert_type(u & _HI, jnp.float32)


def _node_prep_body(x_ref, aw, ab, bw, bb, dw, db, ew, eb, f1w, f1b, f2w, f2b,
                    ax_o, tdb_o, ex_o):
    x = x_ref[...]
    avg = jnp.sum(x, axis=0, keepdims=True)
    r1 = jax.nn.relu(_dotT(avg, f1w[...]) + f1b[...])
    gate = jax.nn.sigmoid(_dotT(r1, f2w[...]) + f2b[...])
    ax_o[...] = _dotT(x, aw[...]) + ab[...]
    dxv = _dotT(x, dw[...]) + db[...]
    bxv = gate * (_dotT(x, bw[...]) + bb[...])
    exv = _dotT(x, ew[...]) + eb[...]
    tdb_o[...] = _pack2(dxv, bxv)
    ex_o[...] = exv


def _node_prep(x, A_w, A_b, B_w, B_b, D_w, D_b, Ew_w, Ew_b,
               FC1_w, FC1_b, FC2_w, FC2_b):
    return pl.pallas_call(
        _node_prep_body,
        out_shape=(jax.ShapeDtypeStruct((N_NODES, D), jnp.float32),
                   jax.ShapeDtypeStruct((N_NODES, D), jnp.uint32),
                   jax.ShapeDtypeStruct((N_NODES, D), jnp.float32)),
    )(x, A_w, A_b.reshape(1, D), B_w, B_b.reshape(1, D), D_w,
      D_b.reshape(1, D), Ew_w, Ew_b.reshape(1, D), FC1_w,
      FC1_b.reshape(1, HD), FC2_w, FC2_b.reshape(1, D))


def _sc_gather_body(tdb_h, exd_h, src_h, dst_h, gdb_h, ge_h,
                    src_v0, dst_v0, gdb_v0, ge_v0,
                    src_v1, dst_v1, gdb_v1, ge_v1,
                    gsem0, gsem1, wsem0, wsem1):
    wid = lax.axis_index("s") * 2 + lax.axis_index("c")

    bufs = ((src_v0, dst_v0, gdb_v0, ge_v0, gsem0, wsem0),
            (src_v1, dst_v1, gdb_v1, ge_v1, gsem1, wsem1))

    @pl.loop(0, A_STEPS + (A_STEPS % 2), step=2)
    def _(t):
        for j in (0, 1):
            src_v, dst_v, gdb_v, ge_v, gsem, wsem = bufs[j]
            blk = (t + j) * NW + wid

            @pl.when(blk < NBLK_C)
            def _():
                pltpu.sync_copy(src_h.at[blk], src_v)
                pltpu.sync_copy(dst_h.at[blk], dst_v)
                pltpu.async_copy(tdb_h.at[src_v.at[0]], gdb_v, gsem)
                pltpu.async_copy(exd_h.at[dst_v.at[0]], ge_v, gsem)

        for j in (0, 1):
            src_v, dst_v, gdb_v, ge_v, gsem, wsem = bufs[j]
            blk = (t + j) * NW + wid

            @pl.when(blk < NBLK_C)
            def _():
                pltpu.make_async_copy(tdb_h.at[src_v.at[0]], gdb_v, gsem).wait()
                pltpu.make_async_copy(exd_h.at[dst_v.at[0]], ge_v, gsem).wait()
                pltpu.async_copy(gdb_v, gdb_h.at[pl.ds(blk * K, K)], wsem)
                pltpu.async_copy(ge_v, ge_h.at[pl.ds(blk * K, K)], wsem)

        for j in (0, 1):
            src_v, dst_v, gdb_v, ge_v, gsem, wsem = bufs[j]
            blk = (t + j) * NW + wid

            @pl.when(blk < NBLK_C)
            def _():
                pltpu.make_async_copy(gdb_v, gdb_h.at[pl.ds(blk * K, K)],
                                      wsem).wait()
                pltpu.make_async_copy(ge_v, ge_h.at[pl.ds(blk * K, K)],
                                      wsem).wait()


def _sc_gather(tdb, exd, src2, dst2):
    return pl.kernel(
        _sc_gather_body,
        out_type=(jax.ShapeDtypeStruct((E_C, D), jnp.uint32),
                  jax.ShapeDtypeStruct((E_C, D), jnp.float32)),
        mesh=_MESH,
        scratch_types=[pltpu.VMEM((1, K), jnp.int32),
                       pltpu.VMEM((1, K), jnp.int32),
                       pltpu.VMEM((K, D), jnp.uint32),
                       pltpu.VMEM((K, D), jnp.float32),
                       pltpu.VMEM((1, K), jnp.int32),
                       pltpu.VMEM((1, K), jnp.int32),
                       pltpu.VMEM((K, D), jnp.uint32),
                       pltpu.VMEM((K, D), jnp.float32),
                       pltpu.SemaphoreType.DMA,
                       pltpu.SemaphoreType.DMA,
                       pltpu.SemaphoreType.DMA,
                       pltpu.SemaphoreType.DMA],
    )(tdb, exd, src2, dst2)


def _edge_body(ex_ref, gdb_ref, ge_ref, sn_ref, cw, cb,
               ej_o, ms0_o, ms1_o, stat_o, acc):
    i = pl.program_id(0)

    @pl.when(i == 0)
    def _():
        acc[...] = jnp.zeros_like(acc)

    ce = _dotT(ex_ref[...], cw[...]) + cb[...]
    gdb = gdb_ref[...]
    dxs = _unpack_lo(gdb)
    bxs = _unpack_hi(gdb)
    ej = ce + dxs + ge_ref[...]
    sig = jax.nn.sigmoid(ej)
    msg = sig * bxs
    ej_o[...] = ej.astype(jnp.bfloat16)
    ms0_o[...] = jnp.concatenate([msg[:, :HD], sig[:, :HD]], axis=1)
    ms1_o[...] = jnp.concatenate([msg[:, HD:], sig[:, HD:]], axis=1)
    v = ej * sn_ref[...]
    acc[0, :] += jnp.sum(v, axis=0)
    acc[1, :] += jnp.sum(v * v, axis=0)

    @pl.when(i == pl.num_programs(0) - 1)
    def _():
        stat_o[...] = acc[...]


def _edge_body_alias(ex_ref, gdb_ref, ge_ref, sn_ref, cw, cb,
                     ejp_ref,
                     ej_o, ms0_o, ms1_o, stat_o, acc):
    _edge_body(ex_ref, gdb_ref, ge_ref, sn_ref, cw, cb,
               ej_o, ms0_o, ms1_o, stat_o, acc)


def _edge_compute(e_x, GDB, GE, snorm_e, C_w, C_b, chunk, prev):
    off = chunk * NEB_C
    eblk_off = pl.BlockSpec((E_BLK, D), lambda i: (i + off, 0))
    outEjB = jax.ShapeDtypeStruct((E_EDGES, D), jnp.bfloat16)
    outMsF = jax.ShapeDtypeStruct((E_C, D), jnp.float32)
    in_specs = [eblk_off,
                pl.BlockSpec((E_BLK, D), lambda i: (i, 0)),
                pl.BlockSpec((E_BLK, D), lambda i: (i, 0)),
                pl.BlockSpec((E_BLK, 1), lambda i: (i + off, 0)),
                pl.BlockSpec((D, D), lambda i: (0, 0)),
                pl.BlockSpec((1, D), lambda i: (0, 0))]
    args = [e_x, GDB, GE, snorm_e, C_w, C_b.reshape(1, D)]
    if prev is None:
        body, aliases = _edge_body, {}
    else:
        body, aliases = _edge_body_alias, {6: 0}
        in_specs += [pl.BlockSpec(memory_space=pltpu.MemorySpace.HBM)]
        args.append(prev)
    return pl.pallas_call(
        body,
        grid=(NEB_C,),
        in_specs=in_specs,
        out_specs=[eblk_off,
                   pl.BlockSpec((E_BLK, D), lambda i: (i, 0)),
                   pl.BlockSpec((E_BLK, D), lambda i: (i, 0)),
                   pl.BlockSpec((2, D), lambda i: (0, 0))],
        out_shape=(outEjB, outMsF, outMsF,
                   jax.ShapeDtypeStruct((2, D), jnp.float32)),
        scratch_shapes=[pltpu.VMEM((2, D), jnp.float32)],
        input_output_aliases=aliases,
    )(*args)


def _sc_scatter_body(off, ms0_h, ms1_h, dst_h, zer_h, acc0_h, acc1_h,
                     dst_v0, m_v0, dst_v1, m_v1, acc_sh,
                     psem0, psem1, ssem0, ssem1):
    cid = lax.axis_index("c")
    sid = lax.axis_index("s")

    pltpu.sync_copy(zer_h, acc_sh.at[pl.ds(sid * STRIPE, STRIPE)])
    plsc.subcore_barrier()

    bufs = ((dst_v0, m_v0, psem0, ssem0), (dst_v1, m_v1, psem1, ssem1))

    def _accumulate(ms_h):
        @pl.loop(0, C_STEPS + (C_STEPS % 2), step=2)
        def _(t):
            for j in (0, 1):
                dst_v, m_v, psem, ssem = bufs[j]
                loc = (t + j) * NSUB + sid
                blk = loc + off

                @pl.when(loc < NBLK_C)
                def _():
                    pltpu.sync_copy(dst_h.at[pl.ds(blk * K, K)], dst_v)
                    pltpu.async_copy(ms_h.at[pl.ds(loc * K, K)], m_v, psem)

            for j in (0, 1):
                dst_v, m_v, psem, ssem = bufs[j]
                loc = (t + j) * NSUB + sid
                blk = loc + off

                @pl.when(loc < NBLK_C)
                def _():
                    pltpu.make_async_copy(ms_h.at[pl.ds(loc * K, K)], m_v,
                                          psem).wait()
                    pltpu.async_copy(m_v, acc_sh.at[dst_v], ssem, add=True)

            for j in (0, 1):
                dst_v, m_v, psem, ssem = bufs[j]
                loc = (t + j) * NSUB + sid
                blk = loc + off

                @pl.when(loc < NBLK_C)
                def _():
                    pltpu.make_async_copy(m_v, acc_sh.at[dst_v], ssem).wait()

    @pl.when(cid == 0)
    def _():
        _accumulate(ms0_h)

    @pl.when(cid == 1)
    def _():
        _accumulate(ms1_h)

    plsc.subcore_barrier()
    sl = pl.ds(sid * STRIPE, STRIPE)

    @pl.when(cid == 0)
    def _():
        pltpu.sync_copy(acc_sh.at[sl], acc0_h.at[sl])

    @pl.when(cid == 1)
    def _():
        pltpu.sync_copy(acc_sh.at[sl], acc1_h.at[sl])


def _sc_scatter(ms0, ms1, dstf, zeros, chunk):
    import functools as _ft
    outA = jax.ShapeDtypeStruct((N_PAD, D), jnp.float32)
    return pl.kernel(
        _ft.partial(_sc_scatter_body, chunk * NBLK_C),
        out_type=(outA, outA),
        mesh=_MESH,
        scratch_types=[pltpu.VMEM((K,), jnp.int32),
                       pltpu.VMEM((K, D), jnp.float32),
                       pltpu.VMEM((K,), jnp.int32),
                       pltpu.VMEM((K, D), jnp.float32),
                       pltpu.VMEM_SHARED((N_PAD, D), jnp.float32),
                       pltpu.SemaphoreType.DMA,
                       pltpu.SemaphoreType.DMA,
                       pltpu.SemaphoreType.DMA,
                       pltpu.SemaphoreType.DMA],
    )(ms0, ms1, dstf, zeros)


def _h_body(x_ref, ax_ref, acc0_ref, acc1_ref,
            sn_ref, g_ref, b_ref, h_o):
    x = x_ref[...]
    a0 = acc0_ref[...][:N_NODES]
    a1 = acc1_ref[...][:N_NODES]
    num = jnp.concatenate([a0[:, :HD], a1[:, :HD]], axis=1)
    den = jnp.concatenate([a0[:, HD:], a1[:, HD:]], axis=1)
    has_in = den > 0
    h = jnp.where(has_in, ax_ref[...] + num / jnp.where(has_in, den, 1.0), x)
    h = h * sn_ref[...]
    mu = jnp.mean(h, axis=0, keepdims=True)
    var = jnp.mean(h * h, axis=0, keepdims=True) - mu * mu
    h = g_ref[...] * (h - mu) * lax.rsqrt(var + 1e-5) + b_ref[...]
    h_o[...] = x + jax.nn.relu(h)


def _h_final(x, AX, accs, snorm_n, bn_h_g, bn_h_b):
    acc0 = accs[0][0]
    acc1 = accs[0][1]
    for a0, a1 in accs[1:]:
        acc0 = acc0 + a0
        acc1 = acc1 + a1
    return pl.pallas_call(
        _h_body,
        out_shape=jax.ShapeDtypeStruct((N_NODES, D), jnp.float32),
    )(x, AX, acc0, acc1, snorm_n,
      bn_h_g.reshape(1, D), bn_h_b.reshape(1, D))


def _e_body(ej_ref, ex_ref, sn_ref, stat_ref, g_ref, b_ref, e_o):
    s = stat_ref[...]
    mu = s[0:1, :] / E_EDGES
    var = s[1:2, :] / E_EDGES - mu * mu
    v = ej_ref[...].astype(jnp.float32) * sn_ref[...]
    v = g_ref[...] * (v - mu) * lax.rsqrt(var + 1e-5) + b_ref[...]
    e_o[...] = ex_ref[...] + jax.nn.relu(v)


def _e_final(e_j, e_x, snorm_e, stats, bn_e_g, bn_e_b):
    nblk = E_EDGES // E_BLK
    eblk = pl.BlockSpec((E_BLK, D), lambda i: (i, 0))
    return pl.pallas_call(
        _e_body,
        grid=(nblk,),
        in_specs=[eblk, eblk,
                  pl.BlockSpec((E_BLK, 1), lambda i: (i, 0)),
                  pl.BlockSpec((2, D), lambda i: (0, 0)),
                  pl.BlockSpec((1, D), lambda i: (0, 0)),
                  pl.BlockSpec((1, D), lambda i: (0, 0))],
        out_specs=eblk,
        out_shape=jax.ShapeDtypeStruct((E_EDGES, D), jnp.float32),
    )(e_j, e_x, snorm_e, stats, bn_e_g.reshape(1, D), bn_e_b.reshape(1, D))


def kernel(x, e_x, snorm_n, snorm_e, edge_index, A_w, A_b, B_w, B_b, C_w, C_b,
           D_w, D_b, Ew_w, Ew_b, FC1_w, FC1_b, FC2_w, FC2_b,
           bn_h_g, bn_h_b, bn_e_g, bn_e_b):
    srcf = edge_index[0].astype(jnp.int32)
    dstf = edge_index[1].astype(jnp.int32)
    src2 = srcf.reshape(NBLK, 1, K)
    dst2 = dstf.reshape(NBLK, 1, K)
    zeros = jnp.zeros((STRIPE, D), jnp.float32)

    AX, TDB, EXD = _node_prep(x, A_w, A_b, B_w, B_b, D_w, D_b, Ew_w, Ew_b,
                              FC1_w, FC1_b, FC2_w, FC2_b)

    prev = None
    stats = None
    accs = []
    for c in range(CH):
        GDBc, GEc = _sc_gather(TDB, EXD,
                               src2[c * NBLK_C:(c + 1) * NBLK_C],
                               dst2[c * NBLK_C:(c + 1) * NBLK_C])
        e_j, ms0, ms1, stats_c = _edge_compute(e_x, GDBc, GEc, snorm_e,
                                               C_w, C_b, c, prev)
        prev = e_j
        stats = stats_c if stats is None else stats + stats_c
        accs.append(_sc_scatter(ms0, ms1, dstf, zeros, c))

    H = _h_final(x, AX, accs, snorm_n, bn_h_g, bn_h_b)
    E_out = _e_final(e_j, e_x, snorm_e, stats, bn_e_g, bn_e_b)
    return H, E_out


# speedup vs baseline: 1.2542x; 1.2542x over previous
import jax
import jax.numpy as jnp
import numpy as np
from jax import lax
from jax.experimental import pallas as pl
from jax.experimental.pallas import tpu as pltpu
from jax.experimental.pallas import tpu_sc as plsc

N_NODES = 10000
E_EDGES = 320000
D = 128
HD = D // 2
E_BLK = 1280

CH = 2
E_C = E_EDGES // CH
NEB_C = E_C // E_BLK
K = 128
NBLK = E_EDGES // K
NBLK_C = E_C // K
NW = 32
NSUB = 16
A_STEPS = (NBLK_C + NW - 1) // NW
C_STEPS = (NBLK_C + NSUB - 1) // NSUB
N_PAD = 10240
STRIPE = N_PAD // NSUB

_MESH = plsc.VectorSubcoreMesh(core_axis_name="c", subcore_axis_name="s")

_HI = np.uint32(0xFFFF0000)


def _dotT(a, w):
    return lax.dot_general(a, w, (((1,), (1,)), ((), ())),
                           preferred_element_type=jnp.float32)


def _pack2(lo, hi):
    lo_r = lo.astype(jnp.bfloat16).astype(jnp.float32)
    hi_r = hi.astype(jnp.bfloat16).astype(jnp.float32)
    lo_u = lax.shift_right_logical(
        lax.bitcast_convert_type(lo_r, jnp.uint32), np.uint32(16))
    hi_u = lax.bitcast_convert_type(hi_r, jnp.uint32) & _HI
    return lo_u | hi_u


def _unpack_lo(u):
    return lax.bitcast_convert_type(
        lax.shift_left(u, np.uint32(16)), jnp.float32)


def _unpack_hi(u):
    return lax.bitcast_convert_type(u & _HI, jnp.float32)


def _node_prep_body(x_ref, aw, ab, bw, bb, dw, db, ew, eb, f1w, f1b, f2w, f2b,
                    ax_o, tdb_o, ex_o):
    x = x_ref[...]
    avg = jnp.sum(x, axis=0, keepdims=True)
    r1 = jax.nn.relu(_dotT(avg, f1w[...]) + f1b[...])
    gate = jax.nn.sigmoid(_dotT(r1, f2w[...]) + f2b[...])
    ax_o[...] = _dotT(x, aw[...]) + ab[...]
    dxv = _dotT(x, dw[...]) + db[...]
    bxv = gate * (_dotT(x, bw[...]) + bb[...])
    exv = _dotT(x, ew[...]) + eb[...]
    tdb_o[...] = _pack2(dxv, bxv)
    ex_o[...] = exv


def _node_prep(x, A_w, A_b, B_w, B_b, D_w, D_b, Ew_w, Ew_b,
               FC1_w, FC1_b, FC2_w, FC2_b):
    return pl.pallas_call(
        _node_prep_body,
        out_shape=(jax.ShapeDtypeStruct((N_NODES, D), jnp.float32),
                   jax.ShapeDtypeStruct((N_NODES, D), jnp.uint32),
                   jax.ShapeDtypeStruct((N_NODES, D), jnp.float32)),
    )(x, A_w, A_b.reshape(1, D), B_w, B_b.reshape(1, D), D_w,
      D_b.reshape(1, D), Ew_w, Ew_b.reshape(1, D), FC1_w,
      FC1_b.reshape(1, HD), FC2_w, FC2_b.reshape(1, D))


def _sc_gather_body(tdb_h, exd_h, src_h, dst_h, gdb_h, ge_h,
                    src_v0, dst_v0, gdb_v0, ge_v0,
                    src_v1, dst_v1, gdb_v1, ge_v1,
                    gsem0, gsem1, wsem0, wsem1):
    wid = lax.axis_index("s") * 2 + lax.axis_index("c")

    bufs = ((src_v0, dst_v0, gdb_v0, ge_v0, gsem0, wsem0),
            (src_v1, dst_v1, gdb_v1, ge_v1, gsem1, wsem1))

    @pl.loop(0, A_STEPS + (A_STEPS % 2), step=2)
    def _(t):
        for j in (0, 1):
            src_v, dst_v, gdb_v, ge_v, gsem, wsem = bufs[j]
            blk = (t + j) * NW + wid

            @pl.when(blk < NBLK_C)
            def _():
                pltpu.sync_copy(src_h.at[blk], src_v)
                pltpu.sync_copy(dst_h.at[blk], dst_v)
                pltpu.async_copy(tdb_h.at[src_v.at[0]], gdb_v, gsem)
                pltpu.async_copy(exd_h.at[dst_v.at[0]], ge_v, gsem)

        for j in (0, 1):
            src_v, dst_v, gdb_v, ge_v, gsem, wsem = bufs[j]
            blk = (t + j) * NW + wid

            @pl.when(blk < NBLK_C)
            def _():
                pltpu.make_async_copy(tdb_h.at[src_v.at[0]], gdb_v, gsem).wait()
                pltpu.make_async_copy(exd_h.at[dst_v.at[0]], ge_v, gsem).wait()
                pltpu.async_copy(gdb_v, gdb_h.at[pl.ds(blk * K, K)], wsem)
                pltpu.async_copy(ge_v, ge_h.at[pl.ds(blk * K, K)], wsem)

        for j in (0, 1):
            src_v, dst_v, gdb_v, ge_v, gsem, wsem = bufs[j]
            blk = (t + j) * NW + wid

            @pl.when(blk < NBLK_C)
            def _():
                pltpu.make_async_copy(gdb_v, gdb_h.at[pl.ds(blk * K, K)],
                                      wsem).wait()
                pltpu.make_async_copy(ge_v, ge_h.at[pl.ds(blk * K, K)],
                                      wsem).wait()


def _sc_gather(tdb, exd, src2, dst2):
    return pl.kernel(
        _sc_gather_body,
        out_type=(jax.ShapeDtypeStruct((E_C, D), jnp.uint32),
                  jax.ShapeDtypeStruct((E_C, D), jnp.float32)),
        mesh=_MESH,
        scratch_types=[pltpu.VMEM((1, K), jnp.int32),
                       pltpu.VMEM((1, K), jnp.int32),
                       pltpu.VMEM((K, D), jnp.uint32),
                       pltpu.VMEM((K, D), jnp.float32),
                       pltpu.VMEM((1, K), jnp.int32),
                       pltpu.VMEM((1, K), jnp.int32),
                       pltpu.VMEM((K, D), jnp.uint32),
                       pltpu.VMEM((K, D), jnp.float32),
                       pltpu.SemaphoreType.DMA,
                       pltpu.SemaphoreType.DMA,
                       pltpu.SemaphoreType.DMA,
                       pltpu.SemaphoreType.DMA],
    )(tdb, exd, src2, dst2)


def _edge_body(ex_ref, gdb_ref, ge_ref, sn_ref, cw, cb,
               ej_o, ms0_o, ms1_o, stat_o, acc):
    i = pl.program_id(0)

    @pl.when(i == 0)
    def _():
        acc[...] = jnp.zeros_like(acc)

    ce = _dotT(ex_ref[...], cw[...]) + cb[...]
    gdb = gdb_ref[...]
    dxs = _unpack_lo(gdb)
    bxs = _unpack_hi(gdb)
    ej = ce + dxs + ge_ref[...]
    sig = jax.nn.sigmoid(ej)
    msg = sig * bxs
    ej_o[...] = ej.astype(jnp.bfloat16)
    ms0_o[...] = jnp.concatenate([msg[:, :HD], sig[:, :HD]], axis=1)
    ms1_o[...] = jnp.concatenate([msg[:, HD:], sig[:, HD:]], axis=1)
    v = ej * sn_ref[...]
    acc[0, :] += jnp.sum(v, axis=0)
    acc[1, :] += jnp.sum(v * v, axis=0)

    @pl.when(i == pl.num_programs(0) - 1)
    def _():
        stat_o[...] = acc[...]


def _edge_body_alias(ex_ref, gdb_ref, ge_ref, sn_ref, cw, cb,
                     ejp_ref,
                     ej_o, ms0_o, ms1_o, stat_o, acc):
    _edge_body(ex_ref, gdb_ref, ge_ref, sn_ref, cw, cb,
               ej_o, ms0_o, ms1_o, stat_o, acc)


def _edge_compute(e_x, GDB, GE, snorm_e, C_w, C_b, chunk, prev):
    off = chunk * NEB_C
    eblk_off = pl.BlockSpec((E_BLK, D), lambda i: (i + off, 0))
    outEjB = jax.ShapeDtypeStruct((E_EDGES, D), jnp.bfloat16)
    outMsF = jax.ShapeDtypeStruct((E_C, D), jnp.float32)
    in_specs = [eblk_off,
                pl.BlockSpec((E_BLK, D), lambda i: (i, 0)),
                pl.BlockSpec((E_BLK, D), lambda i: (i, 0)),
                pl.BlockSpec((E_BLK, 1), lambda i: (i + off, 0)),
                pl.BlockSpec((D, D), lambda i: (0, 0)),
                pl.BlockSpec((1, D), lambda i: (0, 0))]
    args = [e_x, GDB, GE, snorm_e, C_w, C_b.reshape(1, D)]
    if prev is None:
        body, aliases = _edge_body, {}
    else:
        body, aliases = _edge_body_alias, {6: 0}
        in_specs += [pl.BlockSpec(memory_space=pltpu.MemorySpace.HBM)]
        args.append(prev)
    return pl.pallas_call(
        body,
        grid=(NEB_C,),
        in_specs=in_specs,
        out_specs=[eblk_off,
                   pl.BlockSpec((E_BLK, D), lambda i: (i, 0)),
                   pl.BlockSpec((E_BLK, D), lambda i: (i, 0)),
                   pl.BlockSpec((2, D), lambda i: (0, 0))],
        out_shape=(outEjB, outMsF, outMsF,
                   jax.ShapeDtypeStruct((2, D), jnp.float32)),
        scratch_shapes=[pltpu.VMEM((2, D), jnp.float32)],
        input_output_aliases=aliases,
    )(*args)


def _sc_scatter_body(off, ms0_h, ms1_h, dst_h, zer_h, acc0_h, acc1_h,
                     dst_v0, m_v0, dst_v1, m_v1, acc_sh,
                     psem0, psem1, ssem0, ssem1):
    cid = lax.axis_index("c")
    sid = lax.axis_index("s")

    pltpu.sync_copy(zer_h, acc_sh.at[pl.ds(sid * STRIPE, STRIPE)])
    plsc.subcore_barrier()

    bufs = ((dst_v0, m_v0, psem0, ssem0), (dst_v1, m_v1, psem1, ssem1))

    def _accumulate(ms_h):
        @pl.loop(0, C_STEPS + (C_STEPS % 2), step=2)
        def _(t):
            for j in (0, 1):
                dst_v, m_v, psem, ssem = bufs[j]
                loc = (t + j) * NSUB + sid
                blk = loc + off

                @pl.when(loc < NBLK_C)
                def _():
                    pltpu.sync_copy(dst_h.at[pl.ds(blk * K, K)], dst_v)
                    pltpu.async_copy(ms_h.at[pl.ds(loc * K, K)], m_v, psem)

            for j in (0, 1):
                dst_v, m_v, psem, ssem = bufs[j]
                loc = (t + j) * NSUB + sid
                blk = loc + off

                @pl.when(loc < NBLK_C)
                def _():
                    pltpu.make_async_copy(ms_h.at[pl.ds(loc * K, K)], m_v,
                                          psem).wait()
                    pltpu.async_copy(m_v, acc_sh.at[dst_v], ssem, add=True)

            for j in (0, 1):
                dst_v, m_v, psem, ssem = bufs[j]
                loc = (t + j) * NSUB + sid
                blk = loc + off

                @pl.when(loc < NBLK_C)
                def _():
                    pltpu.make_async_copy(m_v, acc_sh.at[dst_v], ssem).wait()

    @pl.when(cid == 0)
    def _():
        _accumulate(ms0_h)

    @pl.when(cid == 1)
    def _():
        _accumulate(ms1_h)

    plsc.subcore_barrier()
    sl = pl.ds(sid * STRIPE, STRIPE)

    @pl.when(cid == 0)
    def _():
        pltpu.sync_copy(acc_sh.at[sl], acc0_h.at[sl])

    @pl.when(cid == 1)
    def _():
        pltpu.sync_copy(acc_sh.at[sl], acc1_h.at[sl])


def _sc_scatter(ms0, ms1, dstf, zeros, chunk):
    import functools as _ft
    outA = jax.ShapeDtypeStruct((N_PAD, D), jnp.float32)
    return pl.kernel(
        _ft.partial(_sc_scatter_body, chunk * NBLK_C),
        out_type=(outA, outA),
        mesh=_MESH,
        scratch_types=[pltpu.VMEM((K,), jnp.int32),
                       pltpu.VMEM((K, D), jnp.float32),
                       pltpu.VMEM((K,), jnp.int32),
                       pltpu.VMEM((K, D), jnp.float32),
                       pltpu.VMEM_SHARED((N_PAD, D), jnp.float32),
                       pltpu.SemaphoreType.DMA,
                       pltpu.SemaphoreType.DMA,
                       pltpu.SemaphoreType.DMA,
                       pltpu.SemaphoreType.DMA],
    )(ms0, ms1, dstf, zeros)


def _h_body(x_ref, ax_ref, acc0_ref, acc1_ref,
            sn_ref, g_ref, b_ref, h_o):
    x = x_ref[...]
    a0 = acc0_ref[...][:N_NODES]
    a1 = acc1_ref[...][:N_NODES]
    num = jnp.concatenate([a0[:, :HD], a1[:, :HD]], axis=1)
    den = jnp.concatenate([a0[:, HD:], a1[:, HD:]], axis=1)
    has_in = den > 0
    h = jnp.where(has_in, ax_ref[...] + num / jnp.where(has_in, den, 1.0), x)
    h = h * sn_ref[...]
    mu = jnp.mean(h, axis=0, keepdims=True)
    var = jnp.mean(h * h, axis=0, keepdims=True) - mu * mu
    h = g_ref[...] * (h - mu) * lax.rsqrt(var + 1e-5) + b_ref[...]
    h_o[...] = x + jax.nn.relu(h)


def _h_final(x, AX, accs, snorm_n, bn_h_g, bn_h_b):
    acc0 = accs[0][0]
    acc1 = accs[0][1]
    for a0, a1 in accs[1:]:
        acc0 = acc0 + a0
        acc1 = acc1 + a1
    return pl.pallas_call(
        _h_body,
        out_shape=jax.ShapeDtypeStruct((N_NODES, D), jnp.float32),
    )(x, AX, acc0, acc1, snorm_n,
      bn_h_g.reshape(1, D), bn_h_b.reshape(1, D))


def _e_body(ej_ref, ex_ref, sn_ref, stat_ref, g_ref, b_ref, e_o):
    s = stat_ref[...]
    mu = s[0:1, :] / E_EDGES
    var = s[1:2, :] / E_EDGES - mu * mu
    v = ej_ref[...].astype(jnp.float32) * sn_ref[...]
    v = g_ref[...] * (v - mu) * lax.rsqrt(var + 1e-5) + b_ref[...]
    e_o[...] = ex_ref[...] + jax.nn.relu(v)


def _e_final(e_j, e_x, snorm_e, stats, bn_e_g, bn_e_b):
    nblk = E_EDGES // E_BLK
    eblk = pl.BlockSpec((E_BLK, D), lambda i: (i, 0))
    return pl.pallas_call(
        _e_body,
        grid=(nblk,),
        in_specs=[eblk, eblk,
                  pl.BlockSpec((E_BLK, 1), lambda i: (i, 0)),
                  pl.BlockSpec((2, D), lambda i: (0, 0)),
                  pl.BlockSpec((1, D), lambda i: (0, 0)),
                  pl.BlockSpec((1, D), lambda i: (0, 0))],
        out_specs=eblk,
        out_shape=jax.ShapeDtypeStruct((E_EDGES, D), jnp.float32),
    )(e_j, e_x, snorm_e, stats, bn_e_g.reshape(1, D), bn_e_b.reshape(1, D))


def kernel(x, e_x, snorm_n, snorm_e, edge_index, A_w, A_b, B_w, B_b, C_w, C_b,
           D_w, D_b, Ew_w, Ew_b, FC1_w, FC1_b, FC2_w, FC2_b,
           bn_h_g, bn_h_b, bn_e_g, bn_e_b):
    srcf = edge_index[0].astype(jnp.int32)
    dstf = edge_index[1].astype(jnp.int32)
    src2 = srcf.reshape(NBLK, 1, K)
    dst2 = dstf.reshape(NBLK, 1, K)
    zeros = jnp.zeros((STRIPE, D), jnp.float32)

    AX, TDB, EXD = _node_prep(x, A_w, A_b, B_w, B_b, D_w, D_b, Ew_w, Ew_b,
                              FC1_w, FC1_b, FC2_w, FC2_b)

    prev = None
    stats = None
    accs = []
    for c in range(CH):
        GDBc, GEc = _sc_gather(TDB, EXD,
                               src2[c * NBLK_C:(c + 1) * NBLK_C],
                               dst2[c * NBLK_C:(c + 1) * NBLK_C])
        e_j, ms0, ms1, stats_c = _edge_compute(e_x, GDBc, GEc, snorm_e,
                                               C_w, C_b, c, prev)
        prev = e_j
        stats = stats_c if stats is None else stats + stats_c
        accs.append(_sc_scatter(ms0, ms1, dstf, zeros, c))

    H = _h_final(x, AX, accs, snorm_n, bn_h_g, bn_h_b)
    E_out = _e_final(e_j, e_x, snorm_e, stats, bn_e_g, bn_e_b)
    return H, E_out


# speedup vs baseline: 1.3593x; 1.0838x over previous
import jax
import jax.numpy as jnp
import numpy as np
from jax import lax
from jax.experimental import pallas as pl
from jax.experimental.pallas import tpu as pltpu
from jax.experimental.pallas import tpu_sc as plsc

N_NODES = 10000
E_EDGES = 320000
D = 128
HD = D // 2
E_BLK = 3200

CH = 2
E_C = E_EDGES // CH
NEB_C = E_C // E_BLK
K = 128
NBLK = E_EDGES // K
NBLK_C = E_C // K
NW = 32
NSUB = 16
A_STEPS = (NBLK_C + NW - 1) // NW
C_STEPS = (NBLK_C + NSUB - 1) // NSUB
N_PAD = 10240
STRIPE = N_PAD // NSUB

_MESH = plsc.VectorSubcoreMesh(core_axis_name="c", subcore_axis_name="s")

_HI = np.uint32(0xFFFF0000)


def _dotT(a, w):
    return lax.dot_general(a, w, (((1,), (1,)), ((), ())),
                           preferred_element_type=jnp.float32)


def _pack2(lo, hi):
    lo_r = lo.astype(jnp.bfloat16).astype(jnp.float32)
    hi_r = hi.astype(jnp.bfloat16).astype(jnp.float32)
    lo_u = lax.shift_right_logical(
        lax.bitcast_convert_type(lo_r, jnp.uint32), np.uint32(16))
    hi_u = lax.bitcast_convert_type(hi_r, jnp.uint32) & _HI
    return lo_u | hi_u


def _unpack_lo(u):
    return lax.bitcast_convert_type(
        lax.shift_left(u, np.uint32(16)), jnp.float32)


def _unpack_hi(u):
    return lax.bitcast_convert_type(u & _HI, jnp.float32)


def _node_prep_body(x_ref, aw, ab, bw, bb, dw, db, ew, eb, f1w, f1b, f2w, f2b,
                    ax_o, tdb_o, ex_o):
    x = x_ref[...]
    avg = jnp.sum(x, axis=0, keepdims=True)
    r1 = jax.nn.relu(_dotT(avg, f1w[...]) + f1b[...])
    gate = jax.nn.sigmoid(_dotT(r1, f2w[...]) + f2b[...])
    ax_o[...] = _dotT(x, aw[...]) + ab[...]
    dxv = _dotT(x, dw[...]) + db[...]
    bxv = gate * (_dotT(x, bw[...]) + bb[...])
    exv = _dotT(x, ew[...]) + eb[...]
    tdb_o[...] = _pack2(dxv, bxv)
    ex_o[...] = exv


def _node_prep(x, A_w, A_b, B_w, B_b, D_w, D_b, Ew_w, Ew_b,
               FC1_w, FC1_b, FC2_w, FC2_b):
    return pl.pallas_call(
        _node_prep_body,
        out_shape=(jax.ShapeDtypeStruct((N_NODES, D), jnp.float32),
                   jax.ShapeDtypeStruct((N_NODES, D), jnp.uint32),
                   jax.ShapeDtypeStruct((N_NODES, D), jnp.float32)),
    )(x, A_w, A_b.reshape(1, D), B_w, B_b.reshape(1, D), D_w,
      D_b.reshape(1, D), Ew_w, Ew_b.reshape(1, D), FC1_w,
      FC1_b.reshape(1, HD), FC2_w, FC2_b.reshape(1, D))


def _sc_gather_body(tdb_h, exd_h, src_h, dst_h, gdb_h, ge_h,
                    src_v0, dst_v0, gdb_v0, ge_v0,
                    src_v1, dst_v1, gdb_v1, ge_v1,
                    gsem0, gsem1, wsem0, wsem1):
    wid = lax.axis_index("s") * 2 + lax.axis_index("c")

    bufs = ((src_v0, dst_v0, gdb_v0, ge_v0, gsem0, wsem0),
            (src_v1, dst_v1, gdb_v1, ge_v1, gsem1, wsem1))

    @pl.loop(0, A_STEPS + (A_STEPS % 2), step=2)
    def _(t):
        for j in (0, 1):
            src_v, dst_v, gdb_v, ge_v, gsem, wsem = bufs[j]
            blk = (t + j) * NW + wid

            @pl.when(blk < NBLK_C)
            def _():
                pltpu.sync_copy(src_h.at[blk], src_v)
                pltpu.sync_copy(dst_h.at[blk], dst_v)
                pltpu.async_copy(tdb_h.at[src_v.at[0]], gdb_v, gsem)
                pltpu.async_copy(exd_h.at[dst_v.at[0]], ge_v, gsem)

        for j in (0, 1):
            src_v, dst_v, gdb_v, ge_v, gsem, wsem = bufs[j]
            blk = (t + j) * NW + wid

            @pl.when(blk < NBLK_C)
            def _():
                pltpu.make_async_copy(tdb_h.at[src_v.at[0]], gdb_v, gsem).wait()
                pltpu.make_async_copy(exd_h.at[dst_v.at[0]], ge_v, gsem).wait()
                pltpu.async_copy(gdb_v, gdb_h.at[pl.ds(blk * K, K)], wsem)
                pltpu.async_copy(ge_v, ge_h.at[pl.ds(blk * K, K)], wsem)

        for j in (0, 1):
            src_v, dst_v, gdb_v, ge_v, gsem, wsem = bufs[j]
            blk = (t + j) * NW + wid

            @pl.when(blk < NBLK_C)
            def _():
                pltpu.make_async_copy(gdb_v, gdb_h.at[pl.ds(blk * K, K)],
                                      wsem).wait()
                pltpu.make_async_copy(ge_v, ge_h.at[pl.ds(blk * K, K)],
                                      wsem).wait()


def _sc_gather(tdb, exd, src2, dst2):
    return pl.kernel(
        _sc_gather_body,
        out_type=(jax.ShapeDtypeStruct((E_C, D), jnp.uint32),
                  jax.ShapeDtypeStruct((E_C, D), jnp.float32)),
        mesh=_MESH,
        scratch_types=[pltpu.VMEM((1, K), jnp.int32),
                       pltpu.VMEM((1, K), jnp.int32),
                       pltpu.VMEM((K, D), jnp.uint32),
                       pltpu.VMEM((K, D), jnp.float32),
                       pltpu.VMEM((1, K), jnp.int32),
                       pltpu.VMEM((1, K), jnp.int32),
                       pltpu.VMEM((K, D), jnp.uint32),
                       pltpu.VMEM((K, D), jnp.float32),
                       pltpu.SemaphoreType.DMA,
                       pltpu.SemaphoreType.DMA,
                       pltpu.SemaphoreType.DMA,
                       pltpu.SemaphoreType.DMA],
    )(tdb, exd, src2, dst2)


def _edge_body(ex_ref, gdb_ref, ge_ref, sn_ref, cw, cb,
               ej_o, ms0_o, ms1_o, stat_o, acc):
    i = pl.program_id(0)

    @pl.when(i == 0)
    def _():
        acc[...] = jnp.zeros_like(acc)

    ce = _dotT(ex_ref[...], cw[...]) + cb[...]
    gdb = gdb_ref[...]
    dxs = _unpack_lo(gdb)
    bxs = _unpack_hi(gdb)
    ej = ce + dxs + ge_ref[...]
    sig = jax.nn.sigmoid(ej)
    msg = sig * bxs
    ej_o[...] = ej.astype(jnp.bfloat16)
    ms0_o[...] = jnp.concatenate([msg[:, :HD], sig[:, :HD]], axis=1)
    ms1_o[...] = jnp.concatenate([msg[:, HD:], sig[:, HD:]], axis=1)
    v = ej * sn_ref[...]
    acc[0, :] += jnp.sum(v, axis=0)
    acc[1, :] += jnp.sum(v * v, axis=0)

    @pl.when(i == pl.num_programs(0) - 1)
    def _():
        stat_o[...] = acc[...]


def _edge_body_alias(ex_ref, gdb_ref, ge_ref, sn_ref, cw, cb,
                     ejp_ref,
                     ej_o, ms0_o, ms1_o, stat_o, acc):
    _edge_body(ex_ref, gdb_ref, ge_ref, sn_ref, cw, cb,
               ej_o, ms0_o, ms1_o, stat_o, acc)


def _edge_compute(e_x, GDB, GE, snorm_e, C_w, C_b, chunk, prev):
    off = chunk * NEB_C
    eblk_off = pl.BlockSpec((E_BLK, D), lambda i: (i + off, 0))
    outEjB = jax.ShapeDtypeStruct((E_EDGES, D), jnp.bfloat16)
    outMsF = jax.ShapeDtypeStruct((E_C, D), jnp.float32)
    in_specs = [eblk_off,
                pl.BlockSpec((E_BLK, D), lambda i: (i, 0)),
                pl.BlockSpec((E_BLK, D), lambda i: (i, 0)),
                pl.BlockSpec((E_BLK, 1), lambda i: (i + off, 0)),
                pl.BlockSpec((D, D), lambda i: (0, 0)),
                pl.BlockSpec((1, D), lambda i: (0, 0))]
    args = [e_x, GDB, GE, snorm_e, C_w, C_b.reshape(1, D)]
    if prev is None:
        body, aliases = _edge_body, {}
    else:
        body, aliases = _edge_body_alias, {6: 0}
        in_specs += [pl.BlockSpec(memory_space=pltpu.MemorySpace.HBM)]
        args.append(prev)
    return pl.pallas_call(
        body,
        grid=(NEB_C,),
        in_specs=in_specs,
        out_specs=[eblk_off,
                   pl.BlockSpec((E_BLK, D), lambda i: (i, 0)),
                   pl.BlockSpec((E_BLK, D), lambda i: (i, 0)),
                   pl.BlockSpec((2, D), lambda i: (0, 0))],
        out_shape=(outEjB, outMsF, outMsF,
                   jax.ShapeDtypeStruct((2, D), jnp.float32)),
        scratch_shapes=[pltpu.VMEM((2, D), jnp.float32)],
        input_output_aliases=aliases,
    )(*args)


def _sc_scatter_body(off, ms0_h, ms1_h, dst_h, zer_h, acc0_h, acc1_h,
                     dst_v0, m_v0, dst_v1, m_v1, acc_sh,
                     psem0, psem1, ssem0, ssem1):
    cid = lax.axis_index("c")
    sid = lax.axis_index("s")

    pltpu.sync_copy(zer_h, acc_sh.at[pl.ds(sid * STRIPE, STRIPE)])
    plsc.subcore_barrier()

    bufs = ((dst_v0, m_v0, psem0, ssem0), (dst_v1, m_v1, psem1, ssem1))

    def _accumulate(ms_h):
        @pl.loop(0, C_STEPS + (C_STEPS % 2), step=2)
        def _(t):
            for j in (0, 1):
                dst_v, m_v, psem, ssem = bufs[j]
                loc = (t + j) * NSUB + sid
                blk = loc + off

                @pl.when(loc < NBLK_C)
                def _():
                    pltpu.sync_copy(dst_h.at[pl.ds(blk * K, K)], dst_v)
                    pltpu.async_copy(ms_h.at[pl.ds(loc * K, K)], m_v, psem)

            for j in (0, 1):
                dst_v, m_v, psem, ssem = bufs[j]
                loc = (t + j) * NSUB + sid
                blk = loc + off

                @pl.when(loc < NBLK_C)
                def _():
                    pltpu.make_async_copy(ms_h.at[pl.ds(loc * K, K)], m_v,
                                          psem).wait()
                    pltpu.async_copy(m_v, acc_sh.at[dst_v], ssem, add=True)

            for j in (0, 1):
                dst_v, m_v, psem, ssem = bufs[j]
                loc = (t + j) * NSUB + sid
                blk = loc + off

                @pl.when(loc < NBLK_C)
                def _():
                    pltpu.make_async_copy(m_v, acc_sh.at[dst_v], ssem).wait()

    @pl.when(cid == 0)
    def _():
        _accumulate(ms0_h)

    @pl.when(cid == 1)
    def _():
        _accumulate(ms1_h)

    plsc.subcore_barrier()
    sl = pl.ds(sid * STRIPE, STRIPE)

    @pl.when(cid == 0)
    def _():
        pltpu.sync_copy(acc_sh.at[sl], acc0_h.at[sl])

    @pl.when(cid == 1)
    def _():
        pltpu.sync_copy(acc_sh.at[sl], acc1_h.at[sl])


def _sc_scatter(ms0, ms1, dstf, zeros, chunk):
    import functools as _ft
    outA = jax.ShapeDtypeStruct((N_PAD, D), jnp.float32)
    return pl.kernel(
        _ft.partial(_sc_scatter_body, chunk * NBLK_C),
        out_type=(outA, outA),
        mesh=_MESH,
        scratch_types=[pltpu.VMEM((K,), jnp.int32),
                       pltpu.VMEM((K, D), jnp.float32),
                       pltpu.VMEM((K,), jnp.int32),
                       pltpu.VMEM((K, D), jnp.float32),
                       pltpu.VMEM_SHARED((N_PAD, D), jnp.float32),
                       pltpu.SemaphoreType.DMA,
                       pltpu.SemaphoreType.DMA,
                       pltpu.SemaphoreType.DMA,
                       pltpu.SemaphoreType.DMA],
    )(ms0, ms1, dstf, zeros)


def _h_body(x_ref, ax_ref, acc0_ref, acc1_ref,
            sn_ref, g_ref, b_ref, h_o):
    x = x_ref[...]
    a0 = acc0_ref[...][:N_NODES]
    a1 = acc1_ref[...][:N_NODES]
    num = jnp.concatenate([a0[:, :HD], a1[:, :HD]], axis=1)
    den = jnp.concatenate([a0[:, HD:], a1[:, HD:]], axis=1)
    has_in = den > 0
    h = jnp.where(has_in, ax_ref[...] + num / jnp.where(has_in, den, 1.0), x)
    h = h * sn_ref[...]
    mu = jnp.mean(h, axis=0, keepdims=True)
    var = jnp.mean(h * h, axis=0, keepdims=True) - mu * mu
    h = g_ref[...] * (h - mu) * lax.rsqrt(var + 1e-5) + b_ref[...]
    h_o[...] = x + jax.nn.relu(h)


def _h_final(x, AX, accs, snorm_n, bn_h_g, bn_h_b):
    acc0 = accs[0][0]
    acc1 = accs[0][1]
    for a0, a1 in accs[1:]:
        acc0 = acc0 + a0
        acc1 = acc1 + a1
    return pl.pallas_call(
        _h_body,
        out_shape=jax.ShapeDtypeStruct((N_NODES, D), jnp.float32),
    )(x, AX, acc0, acc1, snorm_n,
      bn_h_g.reshape(1, D), bn_h_b.reshape(1, D))


def _e_body(ej_ref, ex_ref, sn_ref, stat_ref, g_ref, b_ref, e_o):
    s = stat_ref[...]
    mu = s[0:1, :] / E_EDGES
    var = s[1:2, :] / E_EDGES - mu * mu
    v = ej_ref[...].astype(jnp.float32) * sn_ref[...]
    v = g_ref[...] * (v - mu) * lax.rsqrt(var + 1e-5) + b_ref[...]
    e_o[...] = ex_ref[...] + jax.nn.relu(v)


def _e_final(e_j, e_x, snorm_e, stats, bn_e_g, bn_e_b):
    nblk = E_EDGES // E_BLK
    eblk = pl.BlockSpec((E_BLK, D), lambda i: (i, 0))
    return pl.pallas_call(
        _e_body,
        grid=(nblk,),
        in_specs=[eblk, eblk,
                  pl.BlockSpec((E_BLK, 1), lambda i: (i, 0)),
                  pl.BlockSpec((2, D), lambda i: (0, 0)),
                  pl.BlockSpec((1, D), lambda i: (0, 0)),
                  pl.BlockSpec((1, D), lambda i: (0, 0))],
        out_specs=eblk,
        out_shape=jax.ShapeDtypeStruct((E_EDGES, D), jnp.float32),
    )(e_j, e_x, snorm_e, stats, bn_e_g.reshape(1, D), bn_e_b.reshape(1, D))


def kernel(x, e_x, snorm_n, snorm_e, edge_index, A_w, A_b, B_w, B_b, C_w, C_b,
           D_w, D_b, Ew_w, Ew_b, FC1_w, FC1_b, FC2_w, FC2_b,
           bn_h_g, bn_h_b, bn_e_g, bn_e_b):
    srcf = edge_index[0].astype(jnp.int32)
    dstf = edge_index[1].astype(jnp.int32)
    src2 = srcf.reshape(NBLK, 1, K)
    dst2 = dstf.reshape(NBLK, 1, K)
    zeros = jnp.zeros((STRIPE, D), jnp.float32)

    AX, TDB, EXD = _node_prep(x, A_w, A_b, B_w, B_b, D_w, D_b, Ew_w, Ew_b,
                              FC1_w, FC1_b, FC2_w, FC2_b)

    prev = None
    stats = None
    accs = []
    for c in range(CH):
        GDBc, GEc = _sc_gather(TDB, EXD,
                               src2[c * NBLK_C:(c + 1) * NBLK_C],
                               dst2[c * NBLK_C:(c + 1) * NBLK_C])
        e_j, ms0, ms1, stats_c = _edge_compute(e_x, GDBc, GEc, snorm_e,
                                               C_w, C_b, c, prev)
        prev = e_j
        stats = stats_c if stats is None else stats + stats_c
        accs.append(_sc_scatter(ms0, ms1, dstf, zeros, c))

    H = _h_final(x, AX, accs, snorm_n, bn_h_g, bn_h_b)
    E_out = _e_final(e_j, e_x, snorm_e, stats, bn_e_g, bn_e_b)
    return H, E_out


# speedup vs baseline: 1.3690x; 1.0071x over previous
import jax
import jax.numpy as jnp
import numpy as np
from jax import lax
from jax.experimental import pallas as pl
from jax.experimental.pallas import tpu as pltpu
from jax.experimental.pallas import tpu_sc as plsc

N_NODES = 10000
E_EDGES = 320000
D = 128
HD = D // 2
E_BLK = 6400

CH = 2
E_C = E_EDGES // CH
NEB_C = E_C // E_BLK
K = 128
NBLK = E_EDGES // K
NBLK_C = E_C // K
NW = 32
NSUB = 16
A_STEPS = (NBLK_C + NW - 1) // NW
C_STEPS = (NBLK_C + NSUB - 1) // NSUB
N_PAD = 10240
STRIPE = N_PAD // NSUB

_MESH = plsc.VectorSubcoreMesh(core_axis_name="c", subcore_axis_name="s")

_HI = np.uint32(0xFFFF0000)


def _dotT(a, w):
    return lax.dot_general(a, w, (((1,), (1,)), ((), ())),
                           preferred_element_type=jnp.float32)


def _pack2(lo, hi):
    lo_r = lo.astype(jnp.bfloat16).astype(jnp.float32)
    hi_r = hi.astype(jnp.bfloat16).astype(jnp.float32)
    lo_u = lax.shift_right_logical(
        lax.bitcast_convert_type(lo_r, jnp.uint32), np.uint32(16))
    hi_u = lax.bitcast_convert_type(hi_r, jnp.uint32) & _HI
    return lo_u | hi_u


def _unpack_lo(u):
    return lax.bitcast_convert_type(
        lax.shift_left(u, np.uint32(16)), jnp.float32)


def _unpack_hi(u):
    return lax.bitcast_convert_type(u & _HI, jnp.float32)


def _node_prep_body(x_ref, aw, ab, bw, bb, dw, db, ew, eb, f1w, f1b, f2w, f2b,
                    ax_o, tdb_o, ex_o):
    x = x_ref[...]
    avg = jnp.sum(x, axis=0, keepdims=True)
    r1 = jax.nn.relu(_dotT(avg, f1w[...]) + f1b[...])
    gate = jax.nn.sigmoid(_dotT(r1, f2w[...]) + f2b[...])
    ax_o[...] = _dotT(x, aw[...]) + ab[...]
    dxv = _dotT(x, dw[...]) + db[...]
    bxv = gate * (_dotT(x, bw[...]) + bb[...])
    exv = _dotT(x, ew[...]) + eb[...]
    tdb_o[...] = _pack2(dxv, bxv)
    ex_o[...] = exv


def _node_prep(x, A_w, A_b, B_w, B_b, D_w, D_b, Ew_w, Ew_b,
               FC1_w, FC1_b, FC2_w, FC2_b):
    return pl.pallas_call(
        _node_prep_body,
        out_shape=(jax.ShapeDtypeStruct((N_NODES, D), jnp.float32),
                   jax.ShapeDtypeStruct((N_NODES, D), jnp.uint32),
                   jax.ShapeDtypeStruct((N_NODES, D), jnp.float32)),
    )(x, A_w, A_b.reshape(1, D), B_w, B_b.reshape(1, D), D_w,
      D_b.reshape(1, D), Ew_w, Ew_b.reshape(1, D), FC1_w,
      FC1_b.reshape(1, HD), FC2_w, FC2_b.reshape(1, D))


def _sc_gather_body(tdb_h, exd_h, src_h, dst_h, gdb_h, ge_h,
                    src_v0, dst_v0, gdb_v0, ge_v0,
                    src_v1, dst_v1, gdb_v1, ge_v1,
                    gsem0, gsem1, wsem0, wsem1):
    wid = lax.axis_index("s") * 2 + lax.axis_index("c")

    bufs = ((src_v0, dst_v0, gdb_v0, ge_v0, gsem0, wsem0),
            (src_v1, dst_v1, gdb_v1, ge_v1, gsem1, wsem1))

    @pl.loop(0, A_STEPS + (A_STEPS % 2), step=2)
    def _(t):
        for j in (0, 1):
            src_v, dst_v, gdb_v, ge_v, gsem, wsem = bufs[j]
            blk = (t + j) * NW + wid

            @pl.when(blk < NBLK_C)
            def _():
                pltpu.sync_copy(src_h.at[blk], src_v)
                pltpu.sync_copy(dst_h.at[blk], dst_v)
                pltpu.async_copy(tdb_h.at[src_v.at[0]], gdb_v, gsem)
                pltpu.async_copy(exd_h.at[dst_v.at[0]], ge_v, gsem)

        for j in (0, 1):
            src_v, dst_v, gdb_v, ge_v, gsem, wsem = bufs[j]
            blk = (t + j) * NW + wid

            @pl.when(blk < NBLK_C)
            def _():
                pltpu.make_async_copy(tdb_h.at[src_v.at[0]], gdb_v, gsem).wait()
                pltpu.make_async_copy(exd_h.at[dst_v.at[0]], ge_v, gsem).wait()
                pltpu.async_copy(gdb_v, gdb_h.at[pl.ds(blk * K, K)], wsem)
                pltpu.async_copy(ge_v, ge_h.at[pl.ds(blk * K, K)], wsem)

        for j in (0, 1):
            src_v, dst_v, gdb_v, ge_v, gsem, wsem = bufs[j]
            blk = (t + j) * NW + wid

            @pl.when(blk < NBLK_C)
            def _():
                pltpu.make_async_copy(gdb_v, gdb_h.at[pl.ds(blk * K, K)],
                                      wsem).wait()
                pltpu.make_async_copy(ge_v, ge_h.at[pl.ds(blk * K, K)],
                                      wsem).wait()


def _sc_gather(tdb, exd, src2, dst2):
    return pl.kernel(
        _sc_gather_body,
        out_type=(jax.ShapeDtypeStruct((E_C, D), jnp.uint32),
                  jax.ShapeDtypeStruct((E_C, D), jnp.float32)),
        mesh=_MESH,
        scratch_types=[pltpu.VMEM((1, K), jnp.int32),
                       pltpu.VMEM((1, K), jnp.int32),
                       pltpu.VMEM((K, D), jnp.uint32),
                       pltpu.VMEM((K, D), jnp.float32),
                       pltpu.VMEM((1, K), jnp.int32),
                       pltpu.VMEM((1, K), jnp.int32),
                       pltpu.VMEM((K, D), jnp.uint32),
                       pltpu.VMEM((K, D), jnp.float32),
                       pltpu.SemaphoreType.DMA,
                       pltpu.SemaphoreType.DMA,
                       pltpu.SemaphoreType.DMA,
                       pltpu.SemaphoreType.DMA],
    )(tdb, exd, src2, dst2)


def _edge_body(ex_ref, gdb_ref, ge_ref, sn_ref, cw, cb,
               ej_o, ms0_o, ms1_o, stat_o, acc):
    i = pl.program_id(0)

    @pl.when(i == 0)
    def _():
        acc[...] = jnp.zeros_like(acc)

    ce = _dotT(ex_ref[...], cw[...]) + cb[...]
    gdb = gdb_ref[...]
    dxs = _unpack_lo(gdb)
    bxs = _unpack_hi(gdb)
    ej = ce + dxs + ge_ref[...]
    sig = jax.nn.sigmoid(ej)
    msg = sig * bxs
    ej_o[...] = ej.astype(jnp.bfloat16)
    ms0_o[...] = jnp.concatenate([msg[:, :HD], sig[:, :HD]], axis=1)
    ms1_o[...] = jnp.concatenate([msg[:, HD:], sig[:, HD:]], axis=1)
    v = ej * sn_ref[...]
    acc[0, :] += jnp.sum(v, axis=0)
    acc[1, :] += jnp.sum(v * v, axis=0)

    @pl.when(i == pl.num_programs(0) - 1)
    def _():
        stat_o[...] = acc[...]


def _edge_body_alias(ex_ref, gdb_ref, ge_ref, sn_ref, cw, cb,
                     ejp_ref,
                     ej_o, ms0_o, ms1_o, stat_o, acc):
    _edge_body(ex_ref, gdb_ref, ge_ref, sn_ref, cw, cb,
               ej_o, ms0_o, ms1_o, stat_o, acc)


def _edge_compute(e_x, GDB, GE, snorm_e, C_w, C_b, chunk, prev):
    off = chunk * NEB_C
    eblk_off = pl.BlockSpec((E_BLK, D), lambda i: (i + off, 0))
    outEjB = jax.ShapeDtypeStruct((E_EDGES, D), jnp.bfloat16)
    outMsF = jax.ShapeDtypeStruct((E_C, D), jnp.float32)
    in_specs = [eblk_off,
                pl.BlockSpec((E_BLK, D), lambda i: (i, 0)),
                pl.BlockSpec((E_BLK, D), lambda i: (i, 0)),
                pl.BlockSpec((E_BLK, 1), lambda i: (i + off, 0)),
                pl.BlockSpec((D, D), lambda i: (0, 0)),
                pl.BlockSpec((1, D), lambda i: (0, 0))]
    args = [e_x, GDB, GE, snorm_e, C_w, C_b.reshape(1, D)]
    if prev is None:
        body, aliases = _edge_body, {}
    else:
        body, aliases = _edge_body_alias, {6: 0}
        in_specs += [pl.BlockSpec(memory_space=pltpu.MemorySpace.HBM)]
        args.append(prev)
    return pl.pallas_call(
        body,
        grid=(NEB_C,),
        in_specs=in_specs,
        out_specs=[eblk_off,
                   pl.BlockSpec((E_BLK, D), lambda i: (i, 0)),
                   pl.BlockSpec((E_BLK, D), lambda i: (i, 0)),
                   pl.BlockSpec((2, D), lambda i: (0, 0))],
        out_shape=(outEjB, outMsF, outMsF,
                   jax.ShapeDtypeStruct((2, D), jnp.float32)),
        scratch_shapes=[pltpu.VMEM((2, D), jnp.float32)],
        input_output_aliases=aliases,
    )(*args)


def _sc_scatter_body(off, ms0_h, ms1_h, dst_h, zer_h, acc0_h, acc1_h,
                     dst_v0, m_v0, dst_v1, m_v1, acc_sh,
                     psem0, psem1, ssem0, ssem1):
    cid = lax.axis_index("c")
    sid = lax.axis_index("s")

    pltpu.sync_copy(zer_h, acc_sh.at[pl.ds(sid * STRIPE, STRIPE)])
    plsc.subcore_barrier()

    bufs = ((dst_v0, m_v0, psem0, ssem0), (dst_v1, m_v1, psem1, ssem1))

    def _accumulate(ms_h):
        @pl.loop(0, C_STEPS + (C_STEPS % 2), step=2)
        def _(t):
            for j in (0, 1):
                dst_v, m_v, psem, ssem = bufs[j]
                loc = (t + j) * NSUB + sid
                blk = loc + off

                @pl.when(loc < NBLK_C)
                def _():
                    pltpu.sync_copy(dst_h.at[pl.ds(blk * K, K)], dst_v)
                    pltpu.async_copy(ms_h.at[pl.ds(loc * K, K)], m_v, psem)

            for j in (0, 1):
                dst_v, m_v, psem, ssem = bufs[j]
                loc = (t + j) * NSUB + sid
                blk = loc + off

                @pl.when(loc < NBLK_C)
                def _():
                    pltpu.make_async_copy(ms_h.at[pl.ds(loc * K, K)], m_v,
                                          psem).wait()
                    pltpu.async_copy(m_v, acc_sh.at[dst_v], ssem, add=True)

            for j in (0, 1):
                dst_v, m_v, psem, ssem = bufs[j]
                loc = (t + j) * NSUB + sid
                blk = loc + off

                @pl.when(loc < NBLK_C)
                def _():
                    pltpu.make_async_copy(m_v, acc_sh.at[dst_v], ssem).wait()

    @pl.when(cid == 0)
    def _():
        _accumulate(ms0_h)

    @pl.when(cid == 1)
    def _():
        _accumulate(ms1_h)

    plsc.subcore_barrier()
    sl = pl.ds(sid * STRIPE, STRIPE)

    @pl.when(cid == 0)
    def _():
        pltpu.sync_copy(acc_sh.at[sl], acc0_h.at[sl])

    @pl.when(cid == 1)
    def _():
        pltpu.sync_copy(acc_sh.at[sl], acc1_h.at[sl])


def _sc_scatter(ms0, ms1, dstf, zeros, chunk):
    import functools as _ft
    outA = jax.ShapeDtypeStruct((N_PAD, D), jnp.float32)
    return pl.kernel(
        _ft.partial(_sc_scatter_body, chunk * NBLK_C),
        out_type=(outA, outA),
        mesh=_MESH,
        scratch_types=[pltpu.VMEM((K,), jnp.int32),
                       pltpu.VMEM((K, D), jnp.float32),
                       pltpu.VMEM((K,), jnp.int32),
                       pltpu.VMEM((K, D), jnp.float32),
                       pltpu.VMEM_SHARED((N_PAD, D), jnp.float32),
                       pltpu.SemaphoreType.DMA,
                       pltpu.SemaphoreType.DMA,
                       pltpu.SemaphoreType.DMA,
                       pltpu.SemaphoreType.DMA],
    )(ms0, ms1, dstf, zeros)


def _h_body(x_ref, ax_ref, acc0_ref, acc1_ref,
            sn_ref, g_ref, b_ref, h_o):
    x = x_ref[...]
    a0 = acc0_ref[...][:N_NODES]
    a1 = acc1_ref[...][:N_NODES]
    num = jnp.concatenate([a0[:, :HD], a1[:, :HD]], axis=1)
    den = jnp.concatenate([a0[:, HD:], a1[:, HD:]], axis=1)
    has_in = den > 0
    h = jnp.where(has_in, ax_ref[...] + num / jnp.where(has_in, den, 1.0), x)
    h = h * sn_ref[...]
    mu = jnp.mean(h, axis=0, keepdims=True)
    var = jnp.mean(h * h, axis=0, keepdims=True) - mu * mu
    h = g_ref[...] * (h - mu) * lax.rsqrt(var + 1e-5) + b_ref[...]
    h_o[...] = x + jax.nn.relu(h)


def _h_final(x, AX, accs, snorm_n, bn_h_g, bn_h_b):
    acc0 = accs[0][0]
    acc1 = accs[0][1]
    for a0, a1 in accs[1:]:
        acc0 = acc0 + a0
        acc1 = acc1 + a1
    return pl.pallas_call(
        _h_body,
        out_shape=jax.ShapeDtypeStruct((N_NODES, D), jnp.float32),
    )(x, AX, acc0, acc1, snorm_n,
      bn_h_g.reshape(1, D), bn_h_b.reshape(1, D))


def _e_body(ej_ref, ex_ref, sn_ref, stat_ref, g_ref, b_ref, e_o):
    s = stat_ref[...]
    mu = s[0:1, :] / E_EDGES
    var = s[1:2, :] / E_EDGES - mu * mu
    v = ej_ref[...].astype(jnp.float32) * sn_ref[...]
    v = g_ref[...] * (v - mu) * lax.rsqrt(var + 1e-5) + b_ref[...]
    e_o[...] = ex_ref[...] + jax.nn.relu(v)


def _e_final(e_j, e_x, snorm_e, stats, bn_e_g, bn_e_b):
    nblk = E_EDGES // E_BLK
    eblk = pl.BlockSpec((E_BLK, D), lambda i: (i, 0))
    return pl.pallas_call(
        _e_body,
        grid=(nblk,),
        in_specs=[eblk, eblk,
                  pl.BlockSpec((E_BLK, 1), lambda i: (i, 0)),
                  pl.BlockSpec((2, D), lambda i: (0, 0)),
                  pl.BlockSpec((1, D), lambda i: (0, 0)),
                  pl.BlockSpec((1, D), lambda i: (0, 0))],
        out_specs=eblk,
        out_shape=jax.ShapeDtypeStruct((E_EDGES, D), jnp.float32),
    )(e_j, e_x, snorm_e, stats, bn_e_g.reshape(1, D), bn_e_b.reshape(1, D))


def kernel(x, e_x, snorm_n, snorm_e, edge_index, A_w, A_b, B_w, B_b, C_w, C_b,
           D_w, D_b, Ew_w, Ew_b, FC1_w, FC1_b, FC2_w, FC2_b,
           bn_h_g, bn_h_b, bn_e_g, bn_e_b):
    srcf = edge_index[0].astype(jnp.int32)
    dstf = edge_index[1].astype(jnp.int32)
    src2 = srcf.reshape(NBLK, 1, K)
    dst2 = dstf.reshape(NBLK, 1, K)
    zeros = jnp.zeros((STRIPE, D), jnp.float32)

    AX, TDB, EXD = _node_prep(x, A_w, A_b, B_w, B_b, D_w, D_b, Ew_w, Ew_b,
                              FC1_w, FC1_b, FC2_w, FC2_b)

    prev = None
    stats = None
    accs = []
    for c in range(CH):
        GDBc, GEc = _sc_gather(TDB, EXD,
                               src2[c * NBLK_C:(c + 1) * NBLK_C],
                               dst2[c * NBLK_C:(c + 1) * NBLK_C])
        e_j, ms0, ms1, stats_c = _edge_compute(e_x, GDBc, GEc, snorm_e,
                                               C_w, C_b, c, prev)
        prev = e_j
        stats = stats_c if stats is None else stats + stats_c
        accs.append(_sc_scatter(ms0, ms1, dstf, zeros, c))

    H = _h_final(x, AX, accs, snorm_n, bn_h_g, bn_h_b)
    E_out = _e_final(e_j, e_x, snorm_e, stats, bn_e_g, bn_e_b)
    return H, E_out


# speedup vs baseline: 1.3695x; 1.0004x over previous
import jax
import jax.numpy as jnp
import numpy as np
from jax import lax
from jax.experimental import pallas as pl
from jax.experimental.pallas import tpu as pltpu
from jax.experimental.pallas import tpu_sc as plsc

N_NODES = 10000
E_EDGES = 320000
D = 128
HD = D // 2
E_BLK = 6400

CH = 2
E_C = E_EDGES // CH
NEB_C = E_C // E_BLK
K = 128
NBLK = E_EDGES // K
NBLK_C = E_C // K
NW = 32
NSUB = 16
A_STEPS = (NBLK_C + NW - 1) // NW
C_STEPS = (NBLK_C + NSUB - 1) // NSUB
N_PAD = 10240
STRIPE = N_PAD // NSUB

_MESH = plsc.VectorSubcoreMesh(core_axis_name="c", subcore_axis_name="s")

_HI = np.uint32(0xFFFF0000)


def _dotT(a, w):
    return lax.dot_general(a, w, (((1,), (1,)), ((), ())),
                           preferred_element_type=jnp.float32)


def _pack2(lo, hi):
    lo_r = lo.astype(jnp.bfloat16).astype(jnp.float32)
    hi_r = hi.astype(jnp.bfloat16).astype(jnp.float32)
    lo_u = lax.shift_right_logical(
        lax.bitcast_convert_type(lo_r, jnp.uint32), np.uint32(16))
    hi_u = lax.bitcast_convert_type(hi_r, jnp.uint32) & _HI
    return lo_u | hi_u


def _unpack_lo(u):
    return lax.bitcast_convert_type(
        lax.shift_left(u, np.uint32(16)), jnp.float32)


def _unpack_hi(u):
    return lax.bitcast_convert_type(u & _HI, jnp.float32)


def _node_prep_body(x_ref, aw, ab, bw, bb, dw, db, ew, eb, f1w, f1b, f2w, f2b,
                    ax_o, tdb_o, ex_o):
    x = x_ref[...]
    avg = jnp.sum(x, axis=0, keepdims=True)
    r1 = jax.nn.relu(_dotT(avg, f1w[...]) + f1b[...])
    gate = jax.nn.sigmoid(_dotT(r1, f2w[...]) + f2b[...])
    ax_o[...] = _dotT(x, aw[...]) + ab[...]
    dxv = _dotT(x, dw[...]) + db[...]
    bxv = gate * (_dotT(x, bw[...]) + bb[...])
    exv = _dotT(x, ew[...]) + eb[...]
    tdb_o[...] = _pack2(dxv, bxv)
    ex_o[...] = exv


def _node_prep(x, A_w, A_b, B_w, B_b, D_w, D_b, Ew_w, Ew_b,
               FC1_w, FC1_b, FC2_w, FC2_b):
    return pl.pallas_call(
        _node_prep_body,
        out_shape=(jax.ShapeDtypeStruct((N_NODES, D), jnp.float32),
                   jax.ShapeDtypeStruct((N_NODES, D), jnp.uint32),
                   jax.ShapeDtypeStruct((N_NODES, D), jnp.float32)),
    )(x, A_w, A_b.reshape(1, D), B_w, B_b.reshape(1, D), D_w,
      D_b.reshape(1, D), Ew_w, Ew_b.reshape(1, D), FC1_w,
      FC1_b.reshape(1, HD), FC2_w, FC2_b.reshape(1, D))


def _sc_gather_body(tdb_h, exd_h, src_h, dst_h, gdb_h, ge_h,
                    src_v0, dst_v0, gdb_v0, ge_v0,
                    src_v1, dst_v1, gdb_v1, ge_v1,
                    gsem0, gsem1, wsem0, wsem1):
    wid = lax.axis_index("s") * 2 + lax.axis_index("c")

    bufs = ((src_v0, dst_v0, gdb_v0, ge_v0, gsem0, wsem0),
            (src_v1, dst_v1, gdb_v1, ge_v1, gsem1, wsem1))

    @pl.loop(0, A_STEPS + (A_STEPS % 2), step=2)
    def _(t):
        for j in (0, 1):
            src_v, dst_v, gdb_v, ge_v, gsem, wsem = bufs[j]
            blk = (t + j) * NW + wid

            @pl.when(blk < NBLK_C)
            def _():
                pltpu.sync_copy(src_h.at[blk], src_v)
                pltpu.sync_copy(dst_h.at[blk], dst_v)
                pltpu.async_copy(tdb_h.at[src_v.at[0]], gdb_v, gsem)
                pltpu.async_copy(exd_h.at[dst_v.at[0]], ge_v, gsem)

        for j in (0, 1):
            src_v, dst_v, gdb_v, ge_v, gsem, wsem = bufs[j]
            blk = (t + j) * NW + wid

            @pl.when(blk < NBLK_C)
            def _():
                pltpu.make_async_copy(tdb_h.at[src_v.at[0]], gdb_v, gsem).wait()
                pltpu.make_async_copy(exd_h.at[dst_v.at[0]], ge_v, gsem).wait()
                pltpu.async_copy(gdb_v, gdb_h.at[pl.ds(blk * K, K)], wsem)
                pltpu.async_copy(ge_v, ge_h.at[pl.ds(blk * K, K)], wsem)

        for j in (0, 1):
            src_v, dst_v, gdb_v, ge_v, gsem, wsem = bufs[j]
            blk = (t + j) * NW + wid

            @pl.when(blk < NBLK_C)
            def _():
                pltpu.make_async_copy(gdb_v, gdb_h.at[pl.ds(blk * K, K)],
                                      wsem).wait()
                pltpu.make_async_copy(ge_v, ge_h.at[pl.ds(blk * K, K)],
                                      wsem).wait()


def _sc_gather(tdb, exd, src2, dst2):
    return pl.kernel(
        _sc_gather_body,
        out_type=(jax.ShapeDtypeStruct((E_C, D), jnp.uint32),
                  jax.ShapeDtypeStruct((E_C, D), jnp.float32)),
        mesh=_MESH,
        scratch_types=[pltpu.VMEM((1, K), jnp.int32),
                       pltpu.VMEM((1, K), jnp.int32),
                       pltpu.VMEM((K, D), jnp.uint32),
                       pltpu.VMEM((K, D), jnp.float32),
                       pltpu.VMEM((1, K), jnp.int32),
                       pltpu.VMEM((1, K), jnp.int32),
                       pltpu.VMEM((K, D), jnp.uint32),
                       pltpu.VMEM((K, D), jnp.float32),
                       pltpu.SemaphoreType.DMA,
                       pltpu.SemaphoreType.DMA,
                       pltpu.SemaphoreType.DMA,
                       pltpu.SemaphoreType.DMA],
    )(tdb, exd, src2, dst2)


def _edge_body(ex_ref, gdb_ref, ge_ref, sn_ref, cw, cb,
               ej_o, ms0_o, ms1_o, stat_o, acc):
    i = pl.program_id(0)

    @pl.when(i == 0)
    def _():
        acc[...] = jnp.zeros_like(acc)

    ce = _dotT(ex_ref[...], cw[...]) + cb[...]
    gdb = gdb_ref[...]
    dxs = _unpack_lo(gdb)
    bxs = _unpack_hi(gdb)
    ej = ce + dxs + ge_ref[...]
    sig = jax.nn.sigmoid(ej)
    msg = sig * bxs
    ej_o[...] = ej.astype(jnp.bfloat16)
    ms0_o[...] = jnp.concatenate([msg[:, :HD], sig[:, :HD]], axis=1)
    ms1_o[...] = jnp.concatenate([msg[:, HD:], sig[:, HD:]], axis=1)
    v = ej * sn_ref[...]
    acc[0, :] += jnp.sum(v, axis=0)
    acc[1, :] += jnp.sum(v * v, axis=0)

    @pl.when(i == pl.num_programs(0) - 1)
    def _():
        stat_o[...] = acc[...]


def _edge_body_alias(ex_ref, gdb_ref, ge_ref, sn_ref, cw, cb,
                     ejp_ref,
                     ej_o, ms0_o, ms1_o, stat_o, acc):
    _edge_body(ex_ref, gdb_ref, ge_ref, sn_ref, cw, cb,
               ej_o, ms0_o, ms1_o, stat_o, acc)


def _edge_compute(e_x, GDB, GE, snorm_e, C_w, C_b, chunk, prev):
    off = chunk * NEB_C
    eblk_off = pl.BlockSpec((E_BLK, D), lambda i: (i + off, 0))
    outEjB = jax.ShapeDtypeStruct((E_EDGES, D), jnp.bfloat16)
    outMsF = jax.ShapeDtypeStruct((E_C, D), jnp.float32)
    in_specs = [eblk_off,
                pl.BlockSpec((E_BLK, D), lambda i: (i, 0)),
                pl.BlockSpec((E_BLK, D), lambda i: (i, 0)),
                pl.BlockSpec((E_BLK, 1), lambda i: (i + off, 0)),
                pl.BlockSpec((D, D), lambda i: (0, 0)),
                pl.BlockSpec((1, D), lambda i: (0, 0))]
    args = [e_x, GDB, GE, snorm_e, C_w, C_b.reshape(1, D)]
    if prev is None:
        body, aliases = _edge_body, {}
    else:
        body, aliases = _edge_body_alias, {6: 0}
        in_specs += [pl.BlockSpec(memory_space=pltpu.MemorySpace.HBM)]
        args.append(prev)
    return pl.pallas_call(
        body,
        grid=(NEB_C,),
        in_specs=in_specs,
        out_specs=[eblk_off,
                   pl.BlockSpec((E_BLK, D), lambda i: (i, 0)),
                   pl.BlockSpec((E_BLK, D), lambda i: (i, 0)),
                   pl.BlockSpec((2, D), lambda i: (0, 0))],
        out_shape=(outEjB, outMsF, outMsF,
                   jax.ShapeDtypeStruct((2, D), jnp.float32)),
        scratch_shapes=[pltpu.VMEM((2, D), jnp.float32)],
        input_output_aliases=aliases,
    )(*args)


def _sc_scatter_body(off, ms0_h, ms1_h, dst_h, zer_h, acc0_h, acc1_h,
                     dst_v0, m_v0, dst_v1, m_v1, acc_sh,
                     psem0, psem1, ssem0, ssem1):
    cid = lax.axis_index("c")
    sid = lax.axis_index("s")

    pltpu.sync_copy(zer_h, acc_sh.at[pl.ds(sid * STRIPE, STRIPE)])
    plsc.subcore_barrier()

    bufs = ((dst_v0, m_v0, psem0, ssem0), (dst_v1, m_v1, psem1, ssem1))

    def _accumulate(ms_h):
        @pl.loop(0, C_STEPS + (C_STEPS % 2), step=2)
        def _(t):
            for j in (0, 1):
                dst_v, m_v, psem, ssem = bufs[j]
                loc = (t + j) * NSUB + sid
                blk = loc + off

                @pl.when(loc < NBLK_C)
                def _():
                    pltpu.async_copy(dst_h.at[pl.ds(blk * K, K)], dst_v, psem)
                    pltpu.async_copy(ms_h.at[pl.ds(loc * K, K)], m_v, psem)

            for j in (0, 1):
                dst_v, m_v, psem, ssem = bufs[j]
                loc = (t + j) * NSUB + sid
                blk = loc + off

                @pl.when(loc < NBLK_C)
                def _():
                    pltpu.make_async_copy(dst_h.at[pl.ds(blk * K, K)], dst_v,
                                          psem).wait()
                    pltpu.make_async_copy(ms_h.at[pl.ds(loc * K, K)], m_v,
                                          psem).wait()
                    pltpu.async_copy(m_v, acc_sh.at[dst_v], ssem, add=True)

            for j in (0, 1):
                dst_v, m_v, psem, ssem = bufs[j]
                loc = (t + j) * NSUB + sid
                blk = loc + off

                @pl.when(loc < NBLK_C)
                def _():
                    pltpu.make_async_copy(m_v, acc_sh.at[dst_v], ssem).wait()

    @pl.when(cid == 0)
    def _():
        _accumulate(ms0_h)

    @pl.when(cid == 1)
    def _():
        _accumulate(ms1_h)

    plsc.subcore_barrier()
    sl = pl.ds(sid * STRIPE, STRIPE)

    @pl.when(cid == 0)
    def _():
        pltpu.sync_copy(acc_sh.at[sl], acc0_h.at[sl])

    @pl.when(cid == 1)
    def _():
        pltpu.sync_copy(acc_sh.at[sl], acc1_h.at[sl])


def _sc_scatter(ms0, ms1, dstf, zeros, chunk):
    import functools as _ft
    outA = jax.ShapeDtypeStruct((N_PAD, D), jnp.float32)
    return pl.kernel(
        _ft.partial(_sc_scatter_body, chunk * NBLK_C),
        out_type=(outA, outA),
        mesh=_MESH,
        scratch_types=[pltpu.VMEM((K,), jnp.int32),
                       pltpu.VMEM((K, D), jnp.float32),
                       pltpu.VMEM((K,), jnp.int32),
                       pltpu.VMEM((K, D), jnp.float32),
                       pltpu.VMEM_SHARED((N_PAD, D), jnp.float32),
                       pltpu.SemaphoreType.DMA,
                       pltpu.SemaphoreType.DMA,
                       pltpu.SemaphoreType.DMA,
                       pltpu.SemaphoreType.DMA],
    )(ms0, ms1, dstf, zeros)


def _h_body(x_ref, ax_ref, acc0_ref, acc1_ref,
            sn_ref, g_ref, b_ref, h_o):
    x = x_ref[...]
    a0 = acc0_ref[...][:N_NODES]
    a1 = acc1_ref[...][:N_NODES]
    num = jnp.concatenate([a0[:, :HD], a1[:, :HD]], axis=1)
    den = jnp.concatenate([a0[:, HD:], a1[:, HD:]], axis=1)
    has_in = den > 0
    h = jnp.where(has_in, ax_ref[...] + num / jnp.where(has_in, den, 1.0), x)
    h = h * sn_ref[...]
    mu = jnp.mean(h, axis=0, keepdims=True)
    var = jnp.mean(h * h, axis=0, keepdims=True) - mu * mu
    h = g_ref[...] * (h - mu) * lax.rsqrt(var + 1e-5) + b_ref[...]
    h_o[...] = x + jax.nn.relu(h)


def _h_final(x, AX, accs, snorm_n, bn_h_g, bn_h_b):
    acc0 = accs[0][0]
    acc1 = accs[0][1]
    for a0, a1 in accs[1:]:
        acc0 = acc0 + a0
        acc1 = acc1 + a1
    return pl.pallas_call(
        _h_body,
        out_shape=jax.ShapeDtypeStruct((N_NODES, D), jnp.float32),
    )(x, AX, acc0, acc1, snorm_n,
      bn_h_g.reshape(1, D), bn_h_b.reshape(1, D))


def _e_body(ej_ref, ex_ref, sn_ref, stat_ref, g_ref, b_ref, e_o):
    s = stat_ref[...]
    mu = s[0:1, :] / E_EDGES
    var = s[1:2, :] / E_EDGES - mu * mu
    v = ej_ref[...].astype(jnp.float32) * sn_ref[...]
    v = g_ref[...] * (v - mu) * lax.rsqrt(var + 1e-5) + b_ref[...]
    e_o[...] = ex_ref[...] + jax.nn.relu(v)


def _e_final(e_j, e_x, snorm_e, stats, bn_e_g, bn_e_b):
    nblk = E_EDGES // E_BLK
    eblk = pl.BlockSpec((E_BLK, D), lambda i: (i, 0))
    return pl.pallas_call(
        _e_body,
        grid=(nblk,),
        in_specs=[eblk, eblk,
                  pl.BlockSpec((E_BLK, 1), lambda i: (i, 0)),
                  pl.BlockSpec((2, D), lambda i: (0, 0)),
                  pl.BlockSpec((1, D), lambda i: (0, 0)),
                  pl.BlockSpec((1, D), lambda i: (0, 0))],
        out_specs=eblk,
        out_shape=jax.ShapeDtypeStruct((E_EDGES, D), jnp.float32),
    )(e_j, e_x, snorm_e, stats, bn_e_g.reshape(1, D), bn_e_b.reshape(1, D))


def kernel(x, e_x, snorm_n, snorm_e, edge_index, A_w, A_b, B_w, B_b, C_w, C_b,
           D_w, D_b, Ew_w, Ew_b, FC1_w, FC1_b, FC2_w, FC2_b,
           bn_h_g, bn_h_b, bn_e_g, bn_e_b):
    srcf = edge_index[0].astype(jnp.int32)
    dstf = edge_index[1].astype(jnp.int32)
    src2 = srcf.reshape(NBLK, 1, K)
    dst2 = dstf.reshape(NBLK, 1, K)
    zeros = jnp.zeros((STRIPE, D), jnp.float32)

    AX, TDB, EXD = _node_prep(x, A_w, A_b, B_w, B_b, D_w, D_b, Ew_w, Ew_b,
                              FC1_w, FC1_b, FC2_w, FC2_b)

    prev = None
    stats = None
    accs = []
    for c in range(CH):
        GDBc, GEc = _sc_gather(TDB, EXD,
                               src2[c * NBLK_C:(c + 1) * NBLK_C],
                               dst2[c * NBLK_C:(c + 1) * NBLK_C])
        e_j, ms0, ms1, stats_c = _edge_compute(e_x, GDBc, GEc, snorm_e,
                                               C_w, C_b, c, prev)
        prev = e_j
        stats = stats_c if stats is None else stats + stats_c
        accs.append(_sc_scatter(ms0, ms1, dstf, zeros, c))

    H = _h_final(x, AX, accs, snorm_n, bn_h_g, bn_h_b)
    E_out = _e_final(e_j, e_x, snorm_e, stats, bn_e_g, bn_e_b)
    return H, E_out


# speedup vs baseline: 1.3715x; 1.0014x over previous
import jax
import jax.numpy as jnp
import numpy as np
from jax import lax
from jax.experimental import pallas as pl
from jax.experimental.pallas import tpu as pltpu
from jax.experimental.pallas import tpu_sc as plsc

N_NODES = 10000
E_EDGES = 320000
D = 128
HD = D // 2
E_BLK = 6400

CH = 2
E_C = E_EDGES // CH
NEB_C = E_C // E_BLK
K = 128
NBLK = E_EDGES // K
NBLK_C = E_C // K
NW = 32
NSUB = 16
A_STEPS = (NBLK_C + NW - 1) // NW
C_STEPS = (NBLK_C + NSUB - 1) // NSUB
N_PAD = 10240
STRIPE = N_PAD // NSUB

_MESH = plsc.VectorSubcoreMesh(core_axis_name="c", subcore_axis_name="s")

_HI = np.uint32(0xFFFF0000)


def _dotT(a, w):
    return lax.dot_general(a, w, (((1,), (1,)), ((), ())),
                           preferred_element_type=jnp.float32)


def _pack2(lo, hi):
    lo_r = lo.astype(jnp.bfloat16).astype(jnp.float32)
    hi_r = hi.astype(jnp.bfloat16).astype(jnp.float32)
    lo_u = lax.shift_right_logical(
        lax.bitcast_convert_type(lo_r, jnp.uint32), np.uint32(16))
    hi_u = lax.bitcast_convert_type(hi_r, jnp.uint32) & _HI
    return lo_u | hi_u


def _unpack_lo(u):
    return lax.bitcast_convert_type(
        lax.shift_left(u, np.uint32(16)), jnp.float32)


def _unpack_hi(u):
    return lax.bitcast_convert_type(u & _HI, jnp.float32)


def _node_prep_body(x_ref, aw, ab, bw, bb, dw, db, ew, eb, f1w, f1b, f2w, f2b,
                    ax_o, tdb_o, ex_o):
    x = x_ref[...]
    avg = jnp.sum(x, axis=0, keepdims=True)
    r1 = jax.nn.relu(_dotT(avg, f1w[...]) + f1b[...])
    gate = jax.nn.sigmoid(_dotT(r1, f2w[...]) + f2b[...])
    ax_o[...] = _dotT(x, aw[...]) + ab[...]
    dxv = _dotT(x, dw[...]) + db[...]
    bxv = gate * (_dotT(x, bw[...]) + bb[...])
    exv = _dotT(x, ew[...]) + eb[...]
    tdb_o[...] = _pack2(dxv, bxv)
    ex_o[...] = exv


def _node_prep(x, A_w, A_b, B_w, B_b, D_w, D_b, Ew_w, Ew_b,
               FC1_w, FC1_b, FC2_w, FC2_b):
    return pl.pallas_call(
        _node_prep_body,
        out_shape=(jax.ShapeDtypeStruct((N_NODES, D), jnp.float32),
                   jax.ShapeDtypeStruct((N_NODES, D), jnp.uint32),
                   jax.ShapeDtypeStruct((N_NODES, D), jnp.float32)),
    )(x, A_w, A_b.reshape(1, D), B_w, B_b.reshape(1, D), D_w,
      D_b.reshape(1, D), Ew_w, Ew_b.reshape(1, D), FC1_w,
      FC1_b.reshape(1, HD), FC2_w, FC2_b.reshape(1, D))


def _sc_gather_body(tdb_h, exd_h, src_h, dst_h, gdb_h, ge_h,
                    src_v0, dst_v0, gdb_v0, ge_v0,
                    src_v1, dst_v1, gdb_v1, ge_v1,
                    gsem0, gsem1, wsem0, wsem1):
    wid = lax.axis_index("s") * 2 + lax.axis_index("c")

    bufs = ((src_v0, dst_v0, gdb_v0, ge_v0, gsem0, wsem0),
            (src_v1, dst_v1, gdb_v1, ge_v1, gsem1, wsem1))

    @pl.loop(0, A_STEPS + (A_STEPS % 2), step=2)
    def _(t):
        for j in (0, 1):
            src_v, dst_v, gdb_v, ge_v, gsem, wsem = bufs[j]
            blk = (t + j) * NW + wid

            @pl.when(blk < NBLK_C)
            def _():
                pltpu.async_copy(src_h.at[blk], src_v, gsem)
                pltpu.async_copy(dst_h.at[blk], dst_v, gsem)
                pltpu.make_async_copy(src_h.at[blk], src_v, gsem).wait()
                pltpu.make_async_copy(dst_h.at[blk], dst_v, gsem).wait()
                pltpu.async_copy(tdb_h.at[src_v.at[0]], gdb_v, gsem)
                pltpu.async_copy(exd_h.at[dst_v.at[0]], ge_v, gsem)

        for j in (0, 1):
            src_v, dst_v, gdb_v, ge_v, gsem, wsem = bufs[j]
            blk = (t + j) * NW + wid

            @pl.when(blk < NBLK_C)
            def _():
                pltpu.make_async_copy(tdb_h.at[src_v.at[0]], gdb_v, gsem).wait()
                pltpu.make_async_copy(exd_h.at[dst_v.at[0]], ge_v, gsem).wait()
                pltpu.async_copy(gdb_v, gdb_h.at[pl.ds(blk * K, K)], wsem)
                pltpu.async_copy(ge_v, ge_h.at[pl.ds(blk * K, K)], wsem)

        for j in (0, 1):
            src_v, dst_v, gdb_v, ge_v, gsem, wsem = bufs[j]
            blk = (t + j) * NW + wid

            @pl.when(blk < NBLK_C)
            def _():
                pltpu.make_async_copy(gdb_v, gdb_h.at[pl.ds(blk * K, K)],
                                      wsem).wait()
                pltpu.make_async_copy(ge_v, ge_h.at[pl.ds(blk * K, K)],
                                      wsem).wait()


def _sc_gather(tdb, exd, src2, dst2):
    return pl.kernel(
        _sc_gather_body,
        out_type=(jax.ShapeDtypeStruct((E_C, D), jnp.uint32),
                  jax.ShapeDtypeStruct((E_C, D), jnp.float32)),
        mesh=_MESH,
        scratch_types=[pltpu.VMEM((1, K), jnp.int32),
                       pltpu.VMEM((1, K), jnp.int32),
                       pltpu.VMEM((K, D), jnp.uint32),
                       pltpu.VMEM((K, D), jnp.float32),
                       pltpu.VMEM((1, K), jnp.int32),
                       pltpu.VMEM((1, K), jnp.int32),
                       pltpu.VMEM((K, D), jnp.uint32),
                       pltpu.VMEM((K, D), jnp.float32),
                       pltpu.SemaphoreType.DMA,
                       pltpu.SemaphoreType.DMA,
                       pltpu.SemaphoreType.DMA,
                       pltpu.SemaphoreType.DMA],
    )(tdb, exd, src2, dst2)


def _edge_body(ex_ref, gdb_ref, ge_ref, sn_ref, cw, cb,
               ej_o, ms0_o, ms1_o, stat_o, acc):
    i = pl.program_id(0)

    @pl.when(i == 0)
    def _():
        acc[...] = jnp.zeros_like(acc)

    ce = _dotT(ex_ref[...], cw[...]) + cb[...]
    gdb = gdb_ref[...]
    dxs = _unpack_lo(gdb)
    bxs = _unpack_hi(gdb)
    ej = ce + dxs + ge_ref[...]
    sig = jax.nn.sigmoid(ej)
    msg = sig * bxs
    ej_o[...] = ej.astype(jnp.bfloat16)
    ms0_o[...] = jnp.concatenate([msg[:, :HD], sig[:, :HD]], axis=1)
    ms1_o[...] = jnp.concatenate([msg[:, HD:], sig[:, HD:]], axis=1)
    v = ej * sn_ref[...]
    acc[0, :] += jnp.sum(v, axis=0)
    acc[1, :] += jnp.sum(v * v, axis=0)

    @pl.when(i == pl.num_programs(0) - 1)
    def _():
        stat_o[...] = acc[...]


def _edge_body_alias(ex_ref, gdb_ref, ge_ref, sn_ref, cw, cb,
                     ejp_ref,
                     ej_o, ms0_o, ms1_o, stat_o, acc):
    _edge_body(ex_ref, gdb_ref, ge_ref, sn_ref, cw, cb,
               ej_o, ms0_o, ms1_o, stat_o, acc)


def _edge_compute(e_x, GDB, GE, snorm_e, C_w, C_b, chunk, prev):
    off = chunk * NEB_C
    eblk_off = pl.BlockSpec((E_BLK, D), lambda i: (i + off, 0))
    outEjB = jax.ShapeDtypeStruct((E_EDGES, D), jnp.bfloat16)
    outMsF = jax.ShapeDtypeStruct((E_C, D), jnp.float32)
    in_specs = [eblk_off,
                pl.BlockSpec((E_BLK, D), lambda i: (i, 0)),
                pl.BlockSpec((E_BLK, D), lambda i: (i, 0)),
                pl.BlockSpec((E_BLK, 1), lambda i: (i + off, 0)),
                pl.BlockSpec((D, D), lambda i: (0, 0)),
                pl.BlockSpec((1, D), lambda i: (0, 0))]
    args = [e_x, GDB, GE, snorm_e, C_w, C_b.reshape(1, D)]
    if prev is None:
        body, aliases = _edge_body, {}
    else:
        body, aliases = _edge_body_alias, {6: 0}
        in_specs += [pl.BlockSpec(memory_space=pltpu.MemorySpace.HBM)]
        args.append(prev)
    return pl.pallas_call(
        body,
        grid=(NEB_C,),
        in_specs=in_specs,
        out_specs=[eblk_off,
                   pl.BlockSpec((E_BLK, D), lambda i: (i, 0)),
                   pl.BlockSpec((E_BLK, D), lambda i: (i, 0)),
                   pl.BlockSpec((2, D), lambda i: (0, 0))],
        out_shape=(outEjB, outMsF, outMsF,
                   jax.ShapeDtypeStruct((2, D), jnp.float32)),
        scratch_shapes=[pltpu.VMEM((2, D), jnp.float32)],
        input_output_aliases=aliases,
    )(*args)


def _sc_scatter_body(off, ms0_h, ms1_h, dst_h, zer_h, acc0_h, acc1_h,
                     dst_v0, m_v0, dst_v1, m_v1, acc_sh,
                     psem0, psem1, ssem0, ssem1):
    cid = lax.axis_index("c")
    sid = lax.axis_index("s")

    pltpu.sync_copy(zer_h, acc_sh.at[pl.ds(sid * STRIPE, STRIPE)])
    plsc.subcore_barrier()

    bufs = ((dst_v0, m_v0, psem0, ssem0), (dst_v1, m_v1, psem1, ssem1))

    def _accumulate(ms_h):
        @pl.loop(0, C_STEPS + (C_STEPS % 2), step=2)
        def _(t):
            for j in (0, 1):
                dst_v, m_v, psem, ssem = bufs[j]
                loc = (t + j) * NSUB + sid
                blk = loc + off

                @pl.when(loc < NBLK_C)
                def _():
                    pltpu.async_copy(dst_h.at[pl.ds(blk * K, K)], dst_v, psem)
                    pltpu.async_copy(ms_h.at[pl.ds(loc * K, K)], m_v, psem)

            for j in (0, 1):
                dst_v, m_v, psem, ssem = bufs[j]
                loc = (t + j) * NSUB + sid
                blk = loc + off

                @pl.when(loc < NBLK_C)
                def _():
                    pltpu.make_async_copy(dst_h.at[pl.ds(blk * K, K)], dst_v,
                                          psem).wait()
                    pltpu.make_async_copy(ms_h.at[pl.ds(loc * K, K)], m_v,
                                          psem).wait()
                    pltpu.async_copy(m_v, acc_sh.at[dst_v], ssem, add=True)

            for j in (0, 1):
                dst_v, m_v, psem, ssem = bufs[j]
                loc = (t + j) * NSUB + sid
                blk = loc + off

                @pl.when(loc < NBLK_C)
                def _():
                    pltpu.make_async_copy(m_v, acc_sh.at[dst_v], ssem).wait()

    @pl.when(cid == 0)
    def _():
        _accumulate(ms0_h)

    @pl.when(cid == 1)
    def _():
        _accumulate(ms1_h)

    plsc.subcore_barrier()
    sl = pl.ds(sid * STRIPE, STRIPE)

    @pl.when(cid == 0)
    def _():
        pltpu.sync_copy(acc_sh.at[sl], acc0_h.at[sl])

    @pl.when(cid == 1)
    def _():
        pltpu.sync_copy(acc_sh.at[sl], acc1_h.at[sl])


def _sc_scatter(ms0, ms1, dstf, zeros, chunk):
    import functools as _ft
    outA = jax.ShapeDtypeStruct((N_PAD, D), jnp.float32)
    return pl.kernel(
        _ft.partial(_sc_scatter_body, chunk * NBLK_C),
        out_type=(outA, outA),
        mesh=_MESH,
        scratch_types=[pltpu.VMEM((K,), jnp.int32),
                       pltpu.VMEM((K, D), jnp.float32),
                       pltpu.VMEM((K,), jnp.int32),
                       pltpu.VMEM((K, D), jnp.float32),
                       pltpu.VMEM_SHARED((N_PAD, D), jnp.float32),
                       pltpu.SemaphoreType.DMA,
                       pltpu.SemaphoreType.DMA,
                       pltpu.SemaphoreType.DMA,
                       pltpu.SemaphoreType.DMA],
    )(ms0, ms1, dstf, zeros)


def _h_body(x_ref, ax_ref, acc0_ref, acc1_ref,
            sn_ref, g_ref, b_ref, h_o):
    x = x_ref[...]
    a0 = acc0_ref[...][:N_NODES]
    a1 = acc1_ref[...][:N_NODES]
    num = jnp.concatenate([a0[:, :HD], a1[:, :HD]], axis=1)
    den = jnp.concatenate([a0[:, HD:], a1[:, HD:]], axis=1)
    has_in = den > 0
    h = jnp.where(has_in, ax_ref[...] + num / jnp.where(has_in, den, 1.0), x)
    h = h * sn_ref[...]
    mu = jnp.mean(h, axis=0, keepdims=True)
    var = jnp.mean(h * h, axis=0, keepdims=True) - mu * mu
    h = g_ref[...] * (h - mu) * lax.rsqrt(var + 1e-5) + b_ref[...]
    h_o[...] = x + jax.nn.relu(h)


def _h_final(x, AX, accs, snorm_n, bn_h_g, bn_h_b):
    acc0 = accs[0][0]
    acc1 = accs[0][1]
    for a0, a1 in accs[1:]:
        acc0 = acc0 + a0
        acc1 = acc1 + a1
    return pl.pallas_call(
        _h_body,
        out_shape=jax.ShapeDtypeStruct((N_NODES, D), jnp.float32),
    )(x, AX, acc0, acc1, snorm_n,
      bn_h_g.reshape(1, D), bn_h_b.reshape(1, D))


def _e_body(ej_ref, ex_ref, sn_ref, stat_ref, g_ref, b_ref, e_o):
    s = stat_ref[...]
    mu = s[0:1, :] / E_EDGES
    var = s[1:2, :] / E_EDGES - mu * mu
    v = ej_ref[...].astype(jnp.float32) * sn_ref[...]
    v = g_ref[...] * (v - mu) * lax.rsqrt(var + 1e-5) + b_ref[...]
    e_o[...] = ex_ref[...] + jax.nn.relu(v)


def _e_final(e_j, e_x, snorm_e, stats, bn_e_g, bn_e_b):
    nblk = E_EDGES // E_BLK
    eblk = pl.BlockSpec((E_BLK, D), lambda i: (i, 0))
    return pl.pallas_call(
        _e_body,
        grid=(nblk,),
        in_specs=[eblk, eblk,
                  pl.BlockSpec((E_BLK, 1), lambda i: (i, 0)),
                  pl.BlockSpec((2, D), lambda i: (0, 0)),
                  pl.BlockSpec((1, D), lambda i: (0, 0)),
                  pl.BlockSpec((1, D), lambda i: (0, 0))],
        out_specs=eblk,
        out_shape=jax.ShapeDtypeStruct((E_EDGES, D), jnp.float32),
    )(e_j, e_x, snorm_e, stats, bn_e_g.reshape(1, D), bn_e_b.reshape(1, D))


def kernel(x, e_x, snorm_n, snorm_e, edge_index, A_w, A_b, B_w, B_b, C_w, C_b,
           D_w, D_b, Ew_w, Ew_b, FC1_w, FC1_b, FC2_w, FC2_b,
           bn_h_g, bn_h_b, bn_e_g, bn_e_b):
    srcf = edge_index[0].astype(jnp.int32)
    dstf = edge_index[1].astype(jnp.int32)
    src2 = srcf.reshape(NBLK, 1, K)
    dst2 = dstf.reshape(NBLK, 1, K)
    zeros = jnp.zeros((STRIPE, D), jnp.float32)

    AX, TDB, EXD = _node_prep(x, A_w, A_b, B_w, B_b, D_w, D_b, Ew_w, Ew_b,
                              FC1_w, FC1_b, FC2_w, FC2_b)

    prev = None
    stats = None
    accs = []
    for c in range(CH):
        GDBc, GEc = _sc_gather(TDB, EXD,
                               src2[c * NBLK_C:(c + 1) * NBLK_C],
                               dst2[c * NBLK_C:(c + 1) * NBLK_C])
        e_j, ms0, ms1, stats_c = _edge_compute(e_x, GDBc, GEc, snorm_e,
                                               C_w, C_b, c, prev)
        prev = e_j
        stats = stats_c if stats is None else stats + stats_c
        accs.append(_sc_scatter(ms0, ms1, dstf, zeros, c))

    H = _h_final(x, AX, accs, snorm_n, bn_h_g, bn_h_b)
    E_out = _e_final(e_j, e_x, snorm_e, stats, bn_e_g, bn_e_b)
    return H, E_out
